# baseline (speedup 1.0000x reference)
import functools
import math

import jax
import jax.numpy as jnp
from jax import lax
from jax.experimental import pallas as pl
from jax.experimental.pallas import tpu as pltpu
from jax.experimental.pallas import tpu_sc as plsc

C = 128
H = 128
W = 128
E_RAW = 131072
NHALF = H * W
N = 2 * H * W
NUM_HEADS = 8
DH = C // NUM_HEADS
N_LAYERS = 2
SEARCH_RANGE = 3.0
PCR = (-140.8, -40.0, -3.0, 140.8, 40.0, 1.0)

NC = 2
NS = 16
NW = NC * NS
E_TOT = 2 * E_RAW + N

_SC_MESH = plsc.VectorSubcoreMesh(core_axis_name="c", subcore_axis_name="s")
_BR = 512


def _make_gather3():
    per_w = E_TOT // NW
    CH = 128
    n_ch = per_w // CH

    @functools.partial(
        pl.kernel,
        mesh=_SC_MESH,
        out_type=[jax.ShapeDtypeStruct((E_TOT, C), jnp.float32)] * 3,
        scratch_types=[
            pltpu.VMEM((2 * CH,), jnp.int32),
            pltpu.VMEM((2 * CH,), jnp.int32),
            pltpu.VMEM((CH, C), jnp.float32),
            pltpu.VMEM((CH, C), jnp.float32),
            pltpu.VMEM((CH, C), jnp.float32),
            pltpu.VMEM((CH, C), jnp.float32),
            pltpu.VMEM((CH, C), jnp.float32),
            pltpu.VMEM((CH, C), jnp.float32),
            pltpu.SemaphoreType.DMA,
            pltpu.SemaphoreType.DMA,
        ],
    )
    def gather3(k_hbm, q_hbm, v_hbm, src_hbm, dst_hbm,
                ok_hbm, oq_hbm, ov_hbm, idxs, idxd,
                rk0, rq0, rv0, rk1, rq1, rv1, semA, semB):
        wid = lax.axis_index("s") * NC + lax.axis_index("c")
        base = wid * per_w

        @pl.loop(0, n_ch // 2)
        def _(j):
            off = base + 2 * j * CH
            pltpu.sync_copy(src_hbm.at[pl.ds(off, 2 * CH)], idxs)
            pltpu.sync_copy(dst_hbm.at[pl.ds(off, 2 * CH)], idxd)
            sA, dA = idxs.at[pl.ds(0, CH)], idxd.at[pl.ds(0, CH)]
            sB, dB = idxs.at[pl.ds(CH, CH)], idxd.at[pl.ds(CH, CH)]
            a0 = pltpu.async_copy(k_hbm.at[sA], rk0, semA)
            a1 = pltpu.async_copy(q_hbm.at[dA], rq0, semA)
            a2 = pltpu.async_copy(v_hbm.at[sA], rv0, semA)
            b0 = pltpu.async_copy(k_hbm.at[sB], rk1, semB)
            b1 = pltpu.async_copy(q_hbm.at[dB], rq1, semB)
            b2 = pltpu.async_copy(v_hbm.at[sB], rv1, semB)
            a0.wait()
            a1.wait()
            a2.wait()
            pltpu.sync_copy(rk0, ok_hbm.at[pl.ds(off, CH)])
            pltpu.sync_copy(rq0, oq_hbm.at[pl.ds(off, CH)])
            pltpu.sync_copy(rv0, ov_hbm.at[pl.ds(off, CH)])
            b0.wait()
            b1.wait()
            b2.wait()
            pltpu.sync_copy(rk1, ok_hbm.at[pl.ds(off + CH, CH)])
            pltpu.sync_copy(rq1, oq_hbm.at[pl.ds(off + CH, CH)])
            pltpu.sync_copy(rv1, ov_hbm.at[pl.ds(off + CH, CH)])

    return gather3


_GATHER3 = _make_gather3()


_QR = 8192
_ACC_R = _QR + 16
_CH = 128
_DR = NHALF // 16


def _make_scatter3():
    n1 = E_RAW // NS // _CH
    n2 = NHALF // NS // _CH
    ZSTR = _ACC_R // NS
    OSTR = _QR // NS
    DSTR = _DR // NS

    def _sweep(c, s, dstl_hbm, con_hbm, idxr, buf, xform):
        base1 = c * E_RAW + s * (E_RAW // NS)
        @pl.loop(0, n1)
        def _(i):
            off = base1 + i * _CH
            pltpu.sync_copy(dstl_hbm.at[pl.ds(off, _CH)], idxr)
            pltpu.sync_copy(con_hbm.at[pl.ds(off, _CH)], buf)
            xform()
        base2 = 2 * E_RAW + c * NHALF + s * (NHALF // NS)
        @pl.loop(0, n2)
        def _(i):
            off = base2 + i * _CH
            pltpu.sync_copy(dstl_hbm.at[pl.ds(off, _CH)], idxr)
            pltpu.sync_copy(con_hbm.at[pl.ds(off, _CH)], buf)
            xform()

    @functools.partial(
        pl.kernel,
        mesh=_SC_MESH,
        out_type=jax.ShapeDtypeStruct((N, C), jnp.float32),
        scratch_types=[
            pltpu.VMEM_SHARED((_ACC_R, C), jnp.float32),
            pltpu.VMEM((_CH,), jnp.int32),
            pltpu.VMEM((_CH,), jnp.int32),
            pltpu.VMEM((_CH, C), jnp.float32),
        ],
    )
    def scat_hagg(exv_hbm, dstl_hbm, z_hbm, agg_hbm, acc, idxr, idx, buf):
        c = lax.axis_index("c")
        s = lax.axis_index("s")
        for q in (0, 1):
            pltpu.sync_copy(z_hbm, acc.at[pl.ds(s * ZSTR, ZSTR)])
            plsc.subcore_barrier()

            def hagg_x():
                for t in range(_CH // 16):
                    v = idxr[pl.ds(t * 16, 16)]
                    lo = v - q * _QR
                    ok = jnp.logical_and(lo >= 0, lo < _QR)
                    idx[pl.ds(t * 16, 16)] = jnp.where(ok, lo, _QR)
                pltpu.sync_copy(buf, acc.at[idx], add=True)

            _sweep(c, s, dstl_hbm, exv_hbm, idxr, buf, hagg_x)
            plsc.subcore_barrier()
            pltpu.sync_copy(
                acc.at[pl.ds(s * OSTR, OSTR)],
                agg_hbm.at[pl.ds(c * NHALF + q * _QR + s * OSTR, OSTR)])
            plsc.subcore_barrier()

    @functools.partial(
        pl.kernel,
        mesh=_SC_MESH,
        out_type=jax.ShapeDtypeStruct((N // 16, C), jnp.float32),
        scratch_types=[
            pltpu.VMEM_SHARED((_DR, C), jnp.float32),
            pltpu.VMEM((_CH,), jnp.int32),
            pltpu.VMEM((_CH,), jnp.int32),
            pltpu.VMEM((_CH, C), jnp.float32),
        ],
    )
    def scat_den(exs_hbm, dstl_hbm, z_hbm, den_hbm, accd, idxr, idx, buf):
        c = lax.axis_index("c")
        s = lax.axis_index("s")
        pltpu.sync_copy(z_hbm.at[pl.ds(0, DSTR)], accd.at[pl.ds(s * DSTR, DSTR)])
        plsc.subcore_barrier()

        def den_x():
            for t in range(_CH // 16):
                idx[pl.ds(t * 16, 16)] = lax.shift_right_logical(
                    idxr[pl.ds(t * 16, 16)], 4)
            pltpu.sync_copy(buf, accd.at[idx], add=True)

        _sweep(c, s, dstl_hbm, exs_hbm, idxr, buf, den_x)
        plsc.subcore_barrier()
        pltpu.sync_copy(accd.at[pl.ds(s * DSTR, DSTR)],
                        den_hbm.at[pl.ds(c * _DR + s * DSTR, DSTR)])

    return scat_hagg, scat_den


_SCAT_HAGG, _SCAT_DEN = _make_scatter3()


def _lnk(x, g, b):
    m = x.mean(-1, keepdims=True)
    v = ((x - m) ** 2).mean(-1, keepdims=True)
    return (x - m) / jnp.sqrt(v + 1e-5) * g + b


def _headmat():
    ch = lax.broadcasted_iota(jnp.int32, (C, NUM_HEADS), 0) // DH
    hh = lax.broadcasted_iota(jnp.int32, (C, NUM_HEADS), 1)
    return (ch == hh).astype(jnp.float32)


def _embed_body(ego_ref, neb_ref, scl_ref, wdt_ref, bd_ref, out_ref):
    i = pl.program_id(0)
    g = i // (NHALF // _BR)
    blk = i % (NHALF // _BR)
    scl = scl_ref[...]
    s0 = jnp.where(g == 0, scl[0, 0], scl[0, 2])
    s1 = jnp.where(g == 0, scl[0, 1], scl[0, 3])
    hw = (blk * _BR + lax.broadcasted_iota(jnp.int32, (_BR, 1), 0)).astype(jnp.float32)
    ii = jnp.floor(hw / W) - (H - 1) / 2.0
    jj = jnp.mod(hw, W) - (W - 1) / 2.0
    d = jnp.sqrt(jnp.square(s0 * ii) + jnp.square(s1 * jj))
    cc = lax.broadcasted_iota(jnp.int32, (1, C), 1)
    ce = ((cc // 2) * 2).astype(jnp.float32)
    div = jnp.exp(-ce * (math.log(10000.0) / C))
    arg = d * div
    pe = jnp.where((cc % 2) == 0, jnp.sin(arg), jnp.cos(arg)) / math.sqrt(C)
    x = jnp.where(g == 0, ego_ref[...], neb_ref[...]).T
    out_ref[...] = x + jax.lax.dot(pe, wdt_ref[...]) + bd_ref[...]


def _embed(ego2, neb2, scl, wdt, bd):
    nb = NHALF // _BR
    return pl.pallas_call(
        _embed_body,
        grid=(2 * nb,),
        in_specs=[
            pl.BlockSpec((C, _BR), lambda i: (0, i % (NHALF // _BR))),
            pl.BlockSpec((C, _BR), lambda i: (0, i % (NHALF // _BR))),
            pl.BlockSpec((1, 8), lambda i: (0, 0)),
            pl.BlockSpec((C, C), lambda i: (0, 0)),
            pl.BlockSpec((1, C), lambda i: (0, 0)),
        ],
        out_specs=pl.BlockSpec((_BR, C), lambda i: (i, 0)),
        out_shape=jax.ShapeDtypeStruct((N, C), jnp.float32),
    )(ego2, neb2, scl, wdt, bd)


def _einit_body(ev_ref, scl_ref, wd1_ref, bd1_ref, wd2_ref, bd2_ref,
                wee_ref, bee_ref, out_ref):
    i = pl.program_id(0)
    nb = E_RAW // _BR
    ev = ev_ref[...]
    scl = scl_ref[...]
    neb_area = scl[0, 0]
    ego_area = scl[0, 1]
    dis = ev[:, 0:1]
    t = jax.lax.dot(dis, wd1_ref[...]) + bd1_ref[...]
    delta = jax.lax.dot(t, wd2_ref[...]) + bd2_ref[...]
    delta = delta[:, 0]
    ddd = delta / (ev[:, 0] + 1e-7)
    v0 = (ev[:, 0] + delta) / SEARCH_RANGE
    ddn = delta ** 2 / neb_area
    v1 = (ev[:, 1] + ddn) / (1.0 + ddn)
    ddn2 = delta ** 2 / ego_area
    v1n = (ev[:, 1] * (neb_area / ego_area) + ddn2) / (1.0 + ddn2)
    v2 = (ev[:, 2] + ddd) / (1.0 + ddd)
    v3 = (ev[:, 3] + ddd) / (1.0 + ddd)
    r1 = jnp.stack([v0, v1, v2, v3], axis=-1)
    r2 = jnp.stack([v0, v1n, v2, -v3], axis=-1)
    ones = jnp.ones((_BR,), jnp.float32)
    zer = jnp.zeros((_BR,), jnp.float32)
    rc = jnp.stack([zer, ones, zer, ones], axis=-1)
    vals = jnp.where(i < nb, r1, jnp.where(i < 2 * nb, r2, rc))
    out_ref[...] = jax.lax.dot(vals, wee_ref[...]) + bee_ref[...]


def _einit(edge_vals, scl, p):
    nb = E_RAW // _BR
    return pl.pallas_call(
        _einit_body,
        grid=(E_TOT // _BR,),
        in_specs=[
            pl.BlockSpec((_BR, 4), lambda i: (i % (E_RAW // _BR), 0)),
            pl.BlockSpec((1, 8), lambda i: (0, 0)),
            pl.BlockSpec((1, 8), lambda i: (0, 0)),
            pl.BlockSpec((1, 8), lambda i: (0, 0)),
            pl.BlockSpec((8, 1), lambda i: (0, 0)),
            pl.BlockSpec((1, 1), lambda i: (0, 0)),
            pl.BlockSpec((4, C), lambda i: (0, 0)),
            pl.BlockSpec((1, C), lambda i: (0, 0)),
        ],
        out_specs=pl.BlockSpec((_BR, C), lambda i: (i, 0)),
        out_shape=jax.ShapeDtypeStruct((E_TOT, C), jnp.float32),
    )(edge_vals, scl, p['Wd1'], p['bd1'].reshape(1, 8), p['Wd2'],
      p['bd2'].reshape(1, 1), p['W_ee'], p['b_ee'].reshape(1, C))


def _qkv_body(h_ref, wq_ref, wk_ref, wv_ref, q_ref, k_ref, v_ref):
    hb = h_ref[...]
    q_ref[...] = jax.lax.dot(hb, wq_ref[...])
    k_ref[...] = jax.lax.dot(hb, wk_ref[...])
    v_ref[...] = jax.lax.dot(hb, wv_ref[...])


def _qkv(h, p):
    return pl.pallas_call(
        _qkv_body,
        grid=(N // _BR,),
        in_specs=[pl.BlockSpec((_BR, C), lambda i: (i, 0))] +
                 [pl.BlockSpec((C, C), lambda i: (0, 0))] * 3,
        out_specs=[pl.BlockSpec((_BR, C), lambda i: (i, 0))] * 3,
        out_shape=[jax.ShapeDtypeStruct((N, C), jnp.float32)] * 3,
    )(h, p['Wq'], p['Wk'], p['Wv'])


def _escore_body(e_ref, we_ref, ks_ref, qd_ref, vs_ref, dl_ref,
                 woe_ref, boe_ref, ge1_ref, be1_ref, wfe1_ref, bfe1_ref,
                 wfe2_ref, bfe2_ref, ge2_ref, be2_ref,
                 e2_ref, exv_ref, exs_ref):
    e = e_ref[...]
    pe = jax.lax.dot(e, we_ref[...])
    s = ks_ref[...] * qd_ref[...] * pe * (1.0 / math.sqrt(DH))
    e1 = _lnk(e + jax.lax.dot(s, woe_ref[...]) + boe_ref[...],
              ge1_ref[...], be1_ref[...])
    ff = jnp.maximum(jax.lax.dot(e1, wfe1_ref[...]) + bfe1_ref[...], 0.0)
    e2_ref[...] = _lnk(e1 + jax.lax.dot(ff, wfe2_ref[...]) + bfe2_ref[...],
                       ge2_ref[...], be2_ref[...])
    hm = _headmat()
    logits = jnp.clip(jax.lax.dot(s, hm, precision=jax.lax.Precision.HIGHEST),
                      -5.0, 5.0)
    ex = jnp.exp(logits)
    exf = jax.lax.dot(ex, hm.T, precision=jax.lax.Precision.HIGHEST)
    exv_ref[...] = exf * vs_ref[...]
    sh = (jnp.mod(dl_ref[...], 16) ==
          lax.broadcasted_iota(jnp.int32, (1, 16), 1)).astype(jnp.float32)
    cc = lax.broadcasted_iota(jnp.int32, (16, C), 1)
    rr = lax.broadcasted_iota(jnp.int32, (16, C), 0)
    sp = (cc // 8 == rr).astype(jnp.float32)
    cc8 = lax.broadcasted_iota(jnp.int32, (NUM_HEADS, C), 1)
    rr8 = lax.broadcasted_iota(jnp.int32, (NUM_HEADS, C), 0)
    hp = (jnp.mod(cc8, 8) == rr8).astype(jnp.float32)
    exs_ref[...] = (jax.lax.dot(sh, sp, precision=jax.lax.Precision.HIGHEST) *
                    jax.lax.dot(ex, hp, precision=jax.lax.Precision.HIGHEST))


def _escore(e, ksrc, qdst, vsrc, dl2, p):
    r1 = lambda a: a.reshape(1, -1)
    return pl.pallas_call(
        _escore_body,
        grid=(E_TOT // _BR,),
        in_specs=[
            pl.BlockSpec((_BR, C), lambda i: (i, 0)),
            pl.BlockSpec((C, C), lambda i: (0, 0)),
            pl.BlockSpec((_BR, C), lambda i: (i, 0)),
            pl.BlockSpec((_BR, C), lambda i: (i, 0)),
            pl.BlockSpec((_BR, C), lambda i: (i, 0)),
            pl.BlockSpec((_BR, 1), lambda i: (i, 0)),
            pl.BlockSpec((C, C), lambda i: (0, 0)),
            pl.BlockSpec((1, C), lambda i: (0, 0)),
            pl.BlockSpec((1, C), lambda i: (0, 0)),
            pl.BlockSpec((1, C), lambda i: (0, 0)),
            pl.BlockSpec((C, 2 * C), lambda i: (0, 0)),
            pl.BlockSpec((1, 2 * C), lambda i: (0, 0)),
            pl.BlockSpec((2 * C, C), lambda i: (0, 0)),
            pl.BlockSpec((1, C), lambda i: (0, 0)),
            pl.BlockSpec((1, C), lambda i: (0, 0)),
            pl.BlockSpec((1, C), lambda i: (0, 0)),
        ],
        out_specs=[pl.BlockSpec((_BR, C), lambda i: (i, 0))] * 3,
        out_shape=[jax.ShapeDtypeStruct((E_TOT, C), jnp.float32)] * 3,
    )(e, p['We'], ksrc, qdst, vsrc, dl2,
      p['WoE'], r1(p['boE']), r1(p['lnE1g']), r1(p['lnE1b']),
      p['WfE1'], r1(p['bfE1']), p['WfE2'], r1(p['bfE2']),
      r1(p['lnE2g']), r1(p['lnE2b']))


def _hupd_body(h_ref, agg_ref, den_ref, wo_ref, bo_ref, g1_ref, b1_ref,
               wf1_ref, bf1_ref, wf2_ref, bf2_ref, g2_ref, b2_ref, out_ref):
    den8 = den_ref[...] + 1e-9
    den = jax.lax.dot(den8, _headmat().T,
                      precision=jax.lax.Precision.HIGHEST)
    hagg = agg_ref[...] / den
    h = h_ref[...]
    h1 = _lnk(h + jax.lax.dot(hagg, wo_ref[...]) + bo_ref[...],
              g1_ref[...], b1_ref[...])
    ff = jnp.maximum(jax.lax.dot(h1, wf1_ref[...]) + bf1_ref[...], 0.0)
    out_ref[...] = _lnk(h1 + jax.lax.dot(ff, wf2_ref[...]) + bf2_ref[...],
                        g2_ref[...], b2_ref[...])


def _hupd(h, agg, denp, p):
    r1 = lambda a: a.reshape(1, -1)
    return pl.pallas_call(
        _hupd_body,
        grid=(N // _BR,),
        in_specs=[
            pl.BlockSpec((_BR, C), lambda i: (i, 0)),
            pl.BlockSpec((_BR, C), lambda i: (i, 0)),
            pl.BlockSpec((_BR, NUM_HEADS), lambda i: (i, 0)),
            pl.BlockSpec((C, C), lambda i: (0, 0)),
            pl.BlockSpec((1, C), lambda i: (0, 0)),
            pl.BlockSpec((1, C), lambda i: (0, 0)),
            pl.BlockSpec((1, C), lambda i: (0, 0)),
            pl.BlockSpec((C, 2 * C), lambda i: (0, 0)),
            pl.BlockSpec((1, 2 * C), lambda i: (0, 0)),
            pl.BlockSpec((2 * C, C), lambda i: (0, 0)),
            pl.BlockSpec((1, C), lambda i: (0, 0)),
            pl.BlockSpec((1, C), lambda i: (0, 0)),
            pl.BlockSpec((1, C), lambda i: (0, 0)),
        ],
        out_specs=pl.BlockSpec((_BR, C), lambda i: (i, 0)),
        out_shape=jax.ShapeDtypeStruct((N, C), jnp.float32),
    )(h, agg, denp, p['Wo'], r1(p['bo']), r1(p['ln1g']), r1(p['ln1b']),
      p['Wf1'], r1(p['bf1']), p['Wf2'], r1(p['bf2']),
      r1(p['ln2g']), r1(p['ln2b']))


def _eupd_body(e_ref, eo_ref, wo_ref, bo_ref, g1_ref, b1_ref,
               wf1_ref, bf1_ref, wf2_ref, bf2_ref, g2_ref, b2_ref, out_ref):
    e1 = _lnk(e_ref[...] + jax.lax.dot(eo_ref[...], wo_ref[...]) + bo_ref[...],
              g1_ref[...], b1_ref[...])
    ff = jnp.maximum(jax.lax.dot(e1, wf1_ref[...]) + bf1_ref[...], 0.0)
    out_ref[...] = _lnk(e1 + jax.lax.dot(ff, wf2_ref[...]) + bf2_ref[...],
                        g2_ref[...], b2_ref[...])


def _eupd(e, eout, p):
    r1 = lambda a: a.reshape(1, -1)
    return pl.pallas_call(
        _eupd_body,
        grid=(E_TOT // _BR,),
        in_specs=[
            pl.BlockSpec((_BR, C), lambda i: (i, 0)),
            pl.BlockSpec((_BR, C), lambda i: (i, 0)),
            pl.BlockSpec((C, C), lambda i: (0, 0)),
            pl.BlockSpec((1, C), lambda i: (0, 0)),
            pl.BlockSpec((1, C), lambda i: (0, 0)),
            pl.BlockSpec((1, C), lambda i: (0, 0)),
            pl.BlockSpec((C, 2 * C), lambda i: (0, 0)),
            pl.BlockSpec((1, 2 * C), lambda i: (0, 0)),
            pl.BlockSpec((2 * C, C), lambda i: (0, 0)),
            pl.BlockSpec((1, C), lambda i: (0, 0)),
            pl.BlockSpec((1, C), lambda i: (0, 0)),
            pl.BlockSpec((1, C), lambda i: (0, 0)),
        ],
        out_specs=pl.BlockSpec((_BR, C), lambda i: (i, 0)),
        out_shape=jax.ShapeDtypeStruct((E_TOT, C), jnp.float32),
    )(e, eout, p['WoE'], r1(p['boE']), r1(p['lnE1g']), r1(p['lnE1b']),
      p['WfE1'], r1(p['bfE1']), p['WfE2'], r1(p['bfE2']),
      r1(p['lnE2g']), r1(p['lnE2b']))


def _out_transpose_body(h_ref, o_ref):
    o_ref[...] = h_ref[...].reshape(8, W, C).transpose(2, 0, 1)


def _gt_layer(h, e, src_g, dst_g, dst_l, dl2, zrows, p):
    q, k, v = _qkv(h, p)
    ksrc, qdst, vsrc = _GATHER3(k, q, v, src_g, dst_g)
    e2, exv, exs = _escore(e, ksrc, qdst, vsrc, dl2, p)
    agg = _SCAT_HAGG(exv, dst_l, zrows)
    denp = _SCAT_DEN(exs, dst_l, zrows)
    h2 = _hupd(h, agg, denp.reshape(N, NUM_HEADS), p)
    return h2, e2


def kernel(ego, neb, neb_confidence_map, neb_point_cloud_range, edge_index, edge_vals, params):
    neb_pcr = neb_point_cloud_range
    s0e = (PCR[4] - PCR[1]) / H
    s1e = (PCR[3] - PCR[0]) / W
    s0n = (neb_pcr[4] - neb_pcr[1]) / H
    s1n = (neb_pcr[3] - neb_pcr[0]) / W
    neb_area = s0n * s1n
    ego_area = jnp.float32(s0e * s1e)
    scl_embed = jnp.stack([jnp.float32(s0e), jnp.float32(s1e), s0n, s1n,
                           jnp.float32(0), jnp.float32(0), jnp.float32(0),
                           jnp.float32(0)]).reshape(1, 8).astype(jnp.float32)
    scl_einit = jnp.stack([neb_area, ego_area, jnp.float32(0), jnp.float32(0),
                           jnp.float32(0), jnp.float32(0), jnp.float32(0),
                           jnp.float32(0)]).reshape(1, 8).astype(jnp.float32)

    ei0 = edge_index[0].astype(jnp.int32)
    ei1 = edge_index[1].astype(jnp.int32)
    loop = jnp.arange(N, dtype=jnp.int32)
    ar = jnp.arange(NHALF, dtype=jnp.int32)
    src_g = jnp.concatenate([ei0 + NHALF, ei1, loop])
    dst_g = jnp.concatenate([ei1, ei0 + NHALF, loop])
    dst_l = jnp.concatenate([ei1, ei0, ar, ar])
    dl2 = dst_l.reshape(E_TOT, 1)
    zrows = jnp.zeros((_ACC_R // NS, C), jnp.float32)

    h = _embed(ego.reshape(C, NHALF), neb.reshape(C, NHALF), scl_embed,
               params['W_dis'].T, params['b_dis'].reshape(1, C))
    e = _einit(edge_vals, scl_einit, params)
    for l in range(N_LAYERS):
        h, e = _gt_layer(h, e, src_g, dst_g, dst_l, dl2, zrows,
                         params['layers'][l])

    out = pl.pallas_call(
        _out_transpose_body,
        grid=(H // 8,),
        in_specs=[pl.BlockSpec((8 * W, C), lambda i: (i, 0))],
        out_specs=pl.BlockSpec((C, 8, W), lambda i: (0, i, 0)),
        out_shape=jax.ShapeDtypeStruct((C, H, W), jnp.float32),
    )(h[:NHALF])
    return out


# speedup vs baseline: 1.0324x; 1.0324x over previous
import functools
import math

import jax
import jax.numpy as jnp
from jax import lax
from jax.experimental import pallas as pl
from jax.experimental.pallas import tpu as pltpu
from jax.experimental.pallas import tpu_sc as plsc

C = 128
H = 128
W = 128
E_RAW = 131072
NHALF = H * W
N = 2 * H * W
NUM_HEADS = 8
DH = C // NUM_HEADS
N_LAYERS = 2
SEARCH_RANGE = 3.0
PCR = (-140.8, -40.0, -3.0, 140.8, 40.0, 1.0)

NC = 2
NS = 16
NW = NC * NS
E_TOT = 2 * E_RAW + N

_SC_MESH = plsc.VectorSubcoreMesh(core_axis_name="c", subcore_axis_name="s")
_BR = 512


def _make_gather3():
    per_w = E_TOT // NW
    CH = 128
    n_ch = per_w // CH

    @functools.partial(
        pl.kernel,
        mesh=_SC_MESH,
        out_type=[jax.ShapeDtypeStruct((E_TOT, C), jnp.float32)] * 3,
        scratch_types=[
            pltpu.VMEM((2 * CH,), jnp.int32),
            pltpu.VMEM((2 * CH,), jnp.int32),
            pltpu.VMEM((CH, C), jnp.float32),
            pltpu.VMEM((CH, C), jnp.float32),
            pltpu.VMEM((CH, C), jnp.float32),
            pltpu.VMEM((CH, C), jnp.float32),
            pltpu.VMEM((CH, C), jnp.float32),
            pltpu.VMEM((CH, C), jnp.float32),
            pltpu.SemaphoreType.DMA,
            pltpu.SemaphoreType.DMA,
        ],
    )
    def gather3(k_hbm, q_hbm, v_hbm, src_hbm, dst_hbm,
                ok_hbm, oq_hbm, ov_hbm, idxs, idxd,
                rk0, rq0, rv0, rk1, rq1, rv1, semA, semB):
        wid = lax.axis_index("s") * NC + lax.axis_index("c")
        base = wid * per_w

        @pl.loop(0, n_ch // 2)
        def _(j):
            off = base + 2 * j * CH
            pltpu.sync_copy(src_hbm.at[pl.ds(off, 2 * CH)], idxs)
            pltpu.sync_copy(dst_hbm.at[pl.ds(off, 2 * CH)], idxd)
            sA, dA = idxs.at[pl.ds(0, CH)], idxd.at[pl.ds(0, CH)]
            sB, dB = idxs.at[pl.ds(CH, CH)], idxd.at[pl.ds(CH, CH)]
            a0 = pltpu.async_copy(k_hbm.at[sA], rk0, semA)
            a1 = pltpu.async_copy(q_hbm.at[dA], rq0, semA)
            a2 = pltpu.async_copy(v_hbm.at[sA], rv0, semA)
            b0 = pltpu.async_copy(k_hbm.at[sB], rk1, semB)
            b1 = pltpu.async_copy(q_hbm.at[dB], rq1, semB)
            b2 = pltpu.async_copy(v_hbm.at[sB], rv1, semB)
            a0.wait()
            a1.wait()
            a2.wait()
            pltpu.sync_copy(rk0, ok_hbm.at[pl.ds(off, CH)])
            pltpu.sync_copy(rq0, oq_hbm.at[pl.ds(off, CH)])
            pltpu.sync_copy(rv0, ov_hbm.at[pl.ds(off, CH)])
            b0.wait()
            b1.wait()
            b2.wait()
            pltpu.sync_copy(rk1, ok_hbm.at[pl.ds(off + CH, CH)])
            pltpu.sync_copy(rq1, oq_hbm.at[pl.ds(off + CH, CH)])
            pltpu.sync_copy(rv1, ov_hbm.at[pl.ds(off + CH, CH)])

    return gather3


_GATHER3 = _make_gather3()


_QR = 8192
_ACC_R = _QR + 16
_CH = 128
_DR = NHALF // 16


def _make_scatter3():
    n1 = E_RAW // NS // _CH
    n2 = NHALF // NS // _CH
    ZSTR = _ACC_R // NS
    OSTR = _QR // NS
    DSTR = _DR // NS

    def _sweep(c, s, dstl_hbm, con_hbm, idxr, buf, xform):
        base1 = c * E_RAW + s * (E_RAW // NS)
        @pl.loop(0, n1)
        def _(i):
            off = base1 + i * _CH
            pltpu.sync_copy(dstl_hbm.at[pl.ds(off, _CH)], idxr)
            pltpu.sync_copy(con_hbm.at[pl.ds(off, _CH)], buf)
            xform()
        base2 = 2 * E_RAW + c * NHALF + s * (NHALF // NS)
        @pl.loop(0, n2)
        def _(i):
            off = base2 + i * _CH
            pltpu.sync_copy(dstl_hbm.at[pl.ds(off, _CH)], idxr)
            pltpu.sync_copy(con_hbm.at[pl.ds(off, _CH)], buf)
            xform()

    @functools.partial(
        pl.kernel,
        mesh=_SC_MESH,
        out_type=jax.ShapeDtypeStruct((N, C), jnp.float32),
        scratch_types=[
            pltpu.VMEM_SHARED((_ACC_R, C), jnp.float32),
            pltpu.VMEM((_CH,), jnp.int32),
            pltpu.VMEM((_CH,), jnp.int32),
            pltpu.VMEM((_CH, C), jnp.float32),
        ],
    )
    def scat_hagg(exv_hbm, dstl_hbm, z_hbm, agg_hbm, acc, idxr, idx, buf):
        c = lax.axis_index("c")
        s = lax.axis_index("s")
        for q in (0, 1):
            pltpu.sync_copy(z_hbm, acc.at[pl.ds(s * ZSTR, ZSTR)])
            plsc.subcore_barrier()

            def hagg_x():
                for t in range(_CH // 16):
                    v = idxr[pl.ds(t * 16, 16)]
                    lo = v - q * _QR
                    ok = jnp.logical_and(lo >= 0, lo < _QR)
                    idx[pl.ds(t * 16, 16)] = jnp.where(ok, lo, _QR)
                pltpu.sync_copy(buf, acc.at[idx], add=True)

            _sweep(c, s, dstl_hbm, exv_hbm, idxr, buf, hagg_x)
            plsc.subcore_barrier()
            pltpu.sync_copy(
                acc.at[pl.ds(s * OSTR, OSTR)],
                agg_hbm.at[pl.ds(c * NHALF + q * _QR + s * OSTR, OSTR)])
            plsc.subcore_barrier()

    @functools.partial(
        pl.kernel,
        mesh=_SC_MESH,
        out_type=jax.ShapeDtypeStruct((N // 16, C), jnp.float32),
        scratch_types=[
            pltpu.VMEM_SHARED((_DR, C), jnp.float32),
            pltpu.VMEM((_CH,), jnp.int32),
            pltpu.VMEM((_CH,), jnp.int32),
            pltpu.VMEM((_CH, C), jnp.float32),
        ],
    )
    def scat_den(exs_hbm, dstl_hbm, z_hbm, den_hbm, accd, idxr, idx, buf):
        c = lax.axis_index("c")
        s = lax.axis_index("s")
        pltpu.sync_copy(z_hbm.at[pl.ds(0, DSTR)], accd.at[pl.ds(s * DSTR, DSTR)])
        plsc.subcore_barrier()

        def den_x():
            for t in range(_CH // 16):
                idx[pl.ds(t * 16, 16)] = lax.shift_right_logical(
                    idxr[pl.ds(t * 16, 16)], 4)
            pltpu.sync_copy(buf, accd.at[idx], add=True)

        _sweep(c, s, dstl_hbm, exs_hbm, idxr, buf, den_x)
        plsc.subcore_barrier()
        pltpu.sync_copy(accd.at[pl.ds(s * DSTR, DSTR)],
                        den_hbm.at[pl.ds(c * _DR + s * DSTR, DSTR)])

    return scat_hagg, scat_den


_SCAT_HAGG, _SCAT_DEN = _make_scatter3()


def _lnk(x, g, b):
    m = x.mean(-1, keepdims=True)
    v = ((x - m) ** 2).mean(-1, keepdims=True)
    return (x - m) / jnp.sqrt(v + 1e-5) * g + b


def _headmat():
    ch = lax.broadcasted_iota(jnp.int32, (C, NUM_HEADS), 0) // DH
    hh = lax.broadcasted_iota(jnp.int32, (C, NUM_HEADS), 1)
    return (ch == hh).astype(jnp.float32)


def _embed_body(ego_ref, neb_ref, scl_ref, wdt_ref, bd_ref, out_ref):
    i = pl.program_id(0)
    g = i // (NHALF // _BR)
    blk = i % (NHALF // _BR)
    scl = scl_ref[...]
    s0 = jnp.where(g == 0, scl[0, 0], scl[0, 2])
    s1 = jnp.where(g == 0, scl[0, 1], scl[0, 3])
    hw = (blk * _BR + lax.broadcasted_iota(jnp.int32, (_BR, 1), 0)).astype(jnp.float32)
    ii = jnp.floor(hw / W) - (H - 1) / 2.0
    jj = jnp.mod(hw, W) - (W - 1) / 2.0
    d = jnp.sqrt(jnp.square(s0 * ii) + jnp.square(s1 * jj))
    cc = lax.broadcasted_iota(jnp.int32, (1, C), 1)
    ce = ((cc // 2) * 2).astype(jnp.float32)
    div = jnp.exp(-ce * (math.log(10000.0) / C))
    arg = d * div
    pe = jnp.where((cc % 2) == 0, jnp.sin(arg), jnp.cos(arg)) / math.sqrt(C)
    x = jnp.where(g == 0, ego_ref[...], neb_ref[...]).T
    out_ref[...] = x + jax.lax.dot(pe, wdt_ref[...]) + bd_ref[...]


def _embed(ego2, neb2, scl, wdt, bd):
    nb = NHALF // _BR
    return pl.pallas_call(
        _embed_body,
        grid=(2 * nb,),
        in_specs=[
            pl.BlockSpec((C, _BR), lambda i: (0, i % (NHALF // _BR))),
            pl.BlockSpec((C, _BR), lambda i: (0, i % (NHALF // _BR))),
            pl.BlockSpec((1, 8), lambda i: (0, 0)),
            pl.BlockSpec((C, C), lambda i: (0, 0)),
            pl.BlockSpec((1, C), lambda i: (0, 0)),
        ],
        out_specs=pl.BlockSpec((_BR, C), lambda i: (i, 0)),
        out_shape=jax.ShapeDtypeStruct((N, C), jnp.float32),
    )(ego2, neb2, scl, wdt, bd)


def _einit_body(ev_ref, scl_ref, wd1_ref, bd1_ref, wd2_ref, bd2_ref,
                wee_ref, bee_ref, out_ref):
    i = pl.program_id(0)
    nb = E_RAW // _BR
    ev = ev_ref[...]
    scl = scl_ref[...]
    neb_area = scl[0, 0]
    ego_area = scl[0, 1]
    dis = ev[:, 0:1]
    t = jax.lax.dot(dis, wd1_ref[...]) + bd1_ref[...]
    delta = jax.lax.dot(t, wd2_ref[...]) + bd2_ref[...]
    delta = delta[:, 0]
    ddd = delta / (ev[:, 0] + 1e-7)
    v0 = (ev[:, 0] + delta) / SEARCH_RANGE
    ddn = delta ** 2 / neb_area
    v1 = (ev[:, 1] + ddn) / (1.0 + ddn)
    ddn2 = delta ** 2 / ego_area
    v1n = (ev[:, 1] * (neb_area / ego_area) + ddn2) / (1.0 + ddn2)
    v2 = (ev[:, 2] + ddd) / (1.0 + ddd)
    v3 = (ev[:, 3] + ddd) / (1.0 + ddd)
    r1 = jnp.stack([v0, v1, v2, v3], axis=-1)
    r2 = jnp.stack([v0, v1n, v2, -v3], axis=-1)
    ones = jnp.ones((_BR,), jnp.float32)
    zer = jnp.zeros((_BR,), jnp.float32)
    rc = jnp.stack([zer, ones, zer, ones], axis=-1)
    vals = jnp.where(i < nb, r1, jnp.where(i < 2 * nb, r2, rc))
    out_ref[...] = jax.lax.dot(vals, wee_ref[...]) + bee_ref[...]


def _einit(edge_vals, scl, p):
    nb = E_RAW // _BR
    return pl.pallas_call(
        _einit_body,
        grid=(E_TOT // _BR,),
        in_specs=[
            pl.BlockSpec((_BR, 4), lambda i: (i % (E_RAW // _BR), 0)),
            pl.BlockSpec((1, 8), lambda i: (0, 0)),
            pl.BlockSpec((1, 8), lambda i: (0, 0)),
            pl.BlockSpec((1, 8), lambda i: (0, 0)),
            pl.BlockSpec((8, 1), lambda i: (0, 0)),
            pl.BlockSpec((1, 1), lambda i: (0, 0)),
            pl.BlockSpec((4, C), lambda i: (0, 0)),
            pl.BlockSpec((1, C), lambda i: (0, 0)),
        ],
        out_specs=pl.BlockSpec((_BR, C), lambda i: (i, 0)),
        out_shape=jax.ShapeDtypeStruct((E_TOT, C), jnp.float32),
    )(edge_vals, scl, p['Wd1'], p['bd1'].reshape(1, 8), p['Wd2'],
      p['bd2'].reshape(1, 1), p['W_ee'], p['b_ee'].reshape(1, C))


def _qkv_body(h_ref, wq_ref, wk_ref, wv_ref, q_ref, k_ref, v_ref):
    hb = h_ref[...]
    q_ref[...] = jax.lax.dot(hb, wq_ref[...])
    k_ref[...] = jax.lax.dot(hb, wk_ref[...])
    v_ref[...] = jax.lax.dot(hb, wv_ref[...])


def _qkv(h, p):
    return pl.pallas_call(
        _qkv_body,
        grid=(N // _BR,),
        in_specs=[pl.BlockSpec((_BR, C), lambda i: (i, 0))] +
                 [pl.BlockSpec((C, C), lambda i: (0, 0))] * 3,
        out_specs=[pl.BlockSpec((_BR, C), lambda i: (i, 0))] * 3,
        out_shape=[jax.ShapeDtypeStruct((N, C), jnp.float32)] * 3,
    )(h, p['Wq'], p['Wk'], p['Wv'])


def _escore_body(e_ref, we_ref, ks_ref, qd_ref, vs_ref, dl_ref,
                 eout_ref, exv_ref, exs_ref):
    pe = jax.lax.dot(e_ref[...], we_ref[...])
    s = ks_ref[...] * qd_ref[...] * pe * (1.0 / math.sqrt(DH))
    eout_ref[...] = s
    hm = _headmat()
    logits = jnp.clip(jax.lax.dot(s, hm, precision=jax.lax.Precision.HIGHEST),
                      -5.0, 5.0)
    ex = jnp.exp(logits)
    exf = jax.lax.dot(ex, hm.T, precision=jax.lax.Precision.HIGHEST)
    exv_ref[...] = exf * vs_ref[...]
    sh = (jnp.mod(dl_ref[...], 16) ==
          lax.broadcasted_iota(jnp.int32, (1, 16), 1)).astype(jnp.float32)
    cc = lax.broadcasted_iota(jnp.int32, (16, C), 1)
    rr = lax.broadcasted_iota(jnp.int32, (16, C), 0)
    sp = (cc // 8 == rr).astype(jnp.float32)
    cc8 = lax.broadcasted_iota(jnp.int32, (NUM_HEADS, C), 1)
    rr8 = lax.broadcasted_iota(jnp.int32, (NUM_HEADS, C), 0)
    hp = (jnp.mod(cc8, 8) == rr8).astype(jnp.float32)
    exs_ref[...] = (jax.lax.dot(sh, sp, precision=jax.lax.Precision.HIGHEST) *
                    jax.lax.dot(ex, hp, precision=jax.lax.Precision.HIGHEST))


def _escore(e, ksrc, qdst, vsrc, dl2, p):
    return pl.pallas_call(
        _escore_body,
        grid=(E_TOT // _BR,),
        in_specs=[
            pl.BlockSpec((_BR, C), lambda i: (i, 0)),
            pl.BlockSpec((C, C), lambda i: (0, 0)),
            pl.BlockSpec((_BR, C), lambda i: (i, 0)),
            pl.BlockSpec((_BR, C), lambda i: (i, 0)),
            pl.BlockSpec((_BR, C), lambda i: (i, 0)),
            pl.BlockSpec((_BR, 1), lambda i: (i, 0)),
        ],
        out_specs=[pl.BlockSpec((_BR, C), lambda i: (i, 0))] * 3,
        out_shape=[jax.ShapeDtypeStruct((E_TOT, C), jnp.float32)] * 3,
    )(e, p['We'], ksrc, qdst, vsrc, dl2)


def _hupd_body(h_ref, agg_ref, den_ref, wo_ref, bo_ref, g1_ref, b1_ref,
               wf1_ref, bf1_ref, wf2_ref, bf2_ref, g2_ref, b2_ref, out_ref):
    den8 = den_ref[...] + 1e-9
    den = jax.lax.dot(den8, _headmat().T,
                      precision=jax.lax.Precision.HIGHEST)
    hagg = agg_ref[...] / den
    h = h_ref[...]
    h1 = _lnk(h + jax.lax.dot(hagg, wo_ref[...]) + bo_ref[...],
              g1_ref[...], b1_ref[...])
    ff = jnp.maximum(jax.lax.dot(h1, wf1_ref[...]) + bf1_ref[...], 0.0)
    out_ref[...] = _lnk(h1 + jax.lax.dot(ff, wf2_ref[...]) + bf2_ref[...],
                        g2_ref[...], b2_ref[...])


def _hupd(h, agg, denp, p):
    r1 = lambda a: a.reshape(1, -1)
    return pl.pallas_call(
        _hupd_body,
        grid=(N // _BR,),
        in_specs=[
            pl.BlockSpec((_BR, C), lambda i: (i, 0)),
            pl.BlockSpec((_BR, C), lambda i: (i, 0)),
            pl.BlockSpec((_BR, NUM_HEADS), lambda i: (i, 0)),
            pl.BlockSpec((C, C), lambda i: (0, 0)),
            pl.BlockSpec((1, C), lambda i: (0, 0)),
            pl.BlockSpec((1, C), lambda i: (0, 0)),
            pl.BlockSpec((1, C), lambda i: (0, 0)),
            pl.BlockSpec((C, 2 * C), lambda i: (0, 0)),
            pl.BlockSpec((1, 2 * C), lambda i: (0, 0)),
            pl.BlockSpec((2 * C, C), lambda i: (0, 0)),
            pl.BlockSpec((1, C), lambda i: (0, 0)),
            pl.BlockSpec((1, C), lambda i: (0, 0)),
            pl.BlockSpec((1, C), lambda i: (0, 0)),
        ],
        out_specs=pl.BlockSpec((_BR, C), lambda i: (i, 0)),
        out_shape=jax.ShapeDtypeStruct((N, C), jnp.float32),
    )(h, agg, denp, p['Wo'], r1(p['bo']), r1(p['ln1g']), r1(p['ln1b']),
      p['Wf1'], r1(p['bf1']), p['Wf2'], r1(p['bf2']),
      r1(p['ln2g']), r1(p['ln2b']))


def _eupd_body(e_ref, eo_ref, wo_ref, bo_ref, g1_ref, b1_ref,
               wf1_ref, bf1_ref, wf2_ref, bf2_ref, g2_ref, b2_ref, out_ref):
    e1 = _lnk(e_ref[...] + jax.lax.dot(eo_ref[...], wo_ref[...]) + bo_ref[...],
              g1_ref[...], b1_ref[...])
    ff = jnp.maximum(jax.lax.dot(e1, wf1_ref[...]) + bf1_ref[...], 0.0)
    out_ref[...] = _lnk(e1 + jax.lax.dot(ff, wf2_ref[...]) + bf2_ref[...],
                        g2_ref[...], b2_ref[...])


def _eupd(e, eout, p):
    r1 = lambda a: a.reshape(1, -1)
    return pl.pallas_call(
        _eupd_body,
        grid=(E_TOT // _BR,),
        in_specs=[
            pl.BlockSpec((_BR, C), lambda i: (i, 0)),
            pl.BlockSpec((_BR, C), lambda i: (i, 0)),
            pl.BlockSpec((C, C), lambda i: (0, 0)),
            pl.BlockSpec((1, C), lambda i: (0, 0)),
            pl.BlockSpec((1, C), lambda i: (0, 0)),
            pl.BlockSpec((1, C), lambda i: (0, 0)),
            pl.BlockSpec((C, 2 * C), lambda i: (0, 0)),
            pl.BlockSpec((1, 2 * C), lambda i: (0, 0)),
            pl.BlockSpec((2 * C, C), lambda i: (0, 0)),
            pl.BlockSpec((1, C), lambda i: (0, 0)),
            pl.BlockSpec((1, C), lambda i: (0, 0)),
            pl.BlockSpec((1, C), lambda i: (0, 0)),
        ],
        out_specs=pl.BlockSpec((_BR, C), lambda i: (i, 0)),
        out_shape=jax.ShapeDtypeStruct((E_TOT, C), jnp.float32),
    )(e, eout, p['WoE'], r1(p['boE']), r1(p['lnE1g']), r1(p['lnE1b']),
      p['WfE1'], r1(p['bfE1']), p['WfE2'], r1(p['bfE2']),
      r1(p['lnE2g']), r1(p['lnE2b']))


def _out_transpose_body(h_ref, o_ref):
    o_ref[...] = h_ref[...].reshape(8, W, C).transpose(2, 0, 1)


def _gt_layer(h, e, src_g, dst_g, dst_l, dl2, zrows, p):
    q, k, v = _qkv(h, p)
    ksrc, qdst, vsrc = _GATHER3(k, q, v, src_g, dst_g)
    e_out, exv, exs = _escore(e, ksrc, qdst, vsrc, dl2, p)
    agg = _SCAT_HAGG(exv, dst_l, zrows)
    denp = _SCAT_DEN(exs, dst_l, zrows)
    e2 = _eupd(e, e_out, p)
    h2 = _hupd(h, agg, denp.reshape(N, NUM_HEADS), p)
    return h2, e2


def kernel(ego, neb, neb_confidence_map, neb_point_cloud_range, edge_index, edge_vals, params):
    neb_pcr = neb_point_cloud_range
    s0e = (PCR[4] - PCR[1]) / H
    s1e = (PCR[3] - PCR[0]) / W
    s0n = (neb_pcr[4] - neb_pcr[1]) / H
    s1n = (neb_pcr[3] - neb_pcr[0]) / W
    neb_area = s0n * s1n
    ego_area = jnp.float32(s0e * s1e)
    scl_embed = jnp.stack([jnp.float32(s0e), jnp.float32(s1e), s0n, s1n,
                           jnp.float32(0), jnp.float32(0), jnp.float32(0),
                           jnp.float32(0)]).reshape(1, 8).astype(jnp.float32)
    scl_einit = jnp.stack([neb_area, ego_area, jnp.float32(0), jnp.float32(0),
                           jnp.float32(0), jnp.float32(0), jnp.float32(0),
                           jnp.float32(0)]).reshape(1, 8).astype(jnp.float32)

    ei0 = edge_index[0].astype(jnp.int32)
    ei1 = edge_index[1].astype(jnp.int32)
    loop = jnp.arange(N, dtype=jnp.int32)
    ar = jnp.arange(NHALF, dtype=jnp.int32)
    src_g = jnp.concatenate([ei0 + NHALF, ei1, loop])
    dst_g = jnp.concatenate([ei1, ei0 + NHALF, loop])
    dst_l = jnp.concatenate([ei1, ei0, ar, ar])
    dl2 = dst_l.reshape(E_TOT, 1)
    zrows = jnp.zeros((_ACC_R // NS, C), jnp.float32)

    h = _embed(ego.reshape(C, NHALF), neb.reshape(C, NHALF), scl_embed,
               params['W_dis'].T, params['b_dis'].reshape(1, C))
    e = _einit(edge_vals, scl_einit, params)
    for l in range(N_LAYERS):
        h, e = _gt_layer(h, e, src_g, dst_g, dst_l, dl2, zrows,
                         params['layers'][l])

    out = pl.pallas_call(
        _out_transpose_body,
        grid=(H // 8,),
        in_specs=[pl.BlockSpec((8 * W, C), lambda i: (i, 0))],
        out_specs=pl.BlockSpec((C, 8, W), lambda i: (0, i, 0)),
        out_shape=jax.ShapeDtypeStruct((C, H, W), jnp.float32),
    )(h[:NHALF])
    return out


# speedup vs baseline: 1.1173x; 1.0822x over previous
import functools
import math

import jax
import jax.numpy as jnp
from jax import lax
from jax.experimental import pallas as pl
from jax.experimental.pallas import tpu as pltpu
from jax.experimental.pallas import tpu_sc as plsc

C = 128
H = 128
W = 128
E_RAW = 131072
NHALF = H * W
N = 2 * H * W
NUM_HEADS = 8
DH = C // NUM_HEADS
N_LAYERS = 2
SEARCH_RANGE = 3.0
PCR = (-140.8, -40.0, -3.0, 140.8, 40.0, 1.0)

NC = 2
NS = 16
NW = NC * NS
E_TOT = 2 * E_RAW + N

_SC_MESH = plsc.VectorSubcoreMesh(core_axis_name="c", subcore_axis_name="s")
_BR = 512


def _make_gather3():
    per_w = E_TOT // NW
    CH = 128
    n_ch = per_w // CH

    @functools.partial(
        pl.kernel,
        mesh=_SC_MESH,
        out_type=[jax.ShapeDtypeStruct((E_TOT, C), jnp.float32)] * 3,
        scratch_types=[
            pltpu.VMEM((2 * CH,), jnp.int32),
            pltpu.VMEM((2 * CH,), jnp.int32),
            pltpu.VMEM((CH, C), jnp.float32),
            pltpu.VMEM((CH, C), jnp.float32),
            pltpu.VMEM((CH, C), jnp.float32),
            pltpu.VMEM((CH, C), jnp.float32),
            pltpu.VMEM((CH, C), jnp.float32),
            pltpu.VMEM((CH, C), jnp.float32),
            pltpu.SemaphoreType.DMA,
            pltpu.SemaphoreType.DMA,
        ],
    )
    def gather3(k_hbm, q_hbm, v_hbm, src_hbm, dst_hbm,
                ok_hbm, oq_hbm, ov_hbm, idxs, idxd,
                rk0, rq0, rv0, rk1, rq1, rv1, semA, semB):
        wid = lax.axis_index("s") * NC + lax.axis_index("c")
        base = wid * per_w

        @pl.loop(0, n_ch // 2)
        def _(j):
            off = base + 2 * j * CH
            pltpu.sync_copy(src_hbm.at[pl.ds(off, 2 * CH)], idxs)
            pltpu.sync_copy(dst_hbm.at[pl.ds(off, 2 * CH)], idxd)
            sA, dA = idxs.at[pl.ds(0, CH)], idxd.at[pl.ds(0, CH)]
            sB, dB = idxs.at[pl.ds(CH, CH)], idxd.at[pl.ds(CH, CH)]
            a0 = pltpu.async_copy(k_hbm.at[sA], rk0, semA)
            a1 = pltpu.async_copy(q_hbm.at[dA], rq0, semA)
            a2 = pltpu.async_copy(v_hbm.at[sA], rv0, semA)
            b0 = pltpu.async_copy(k_hbm.at[sB], rk1, semB)
            b1 = pltpu.async_copy(q_hbm.at[dB], rq1, semB)
            b2 = pltpu.async_copy(v_hbm.at[sB], rv1, semB)
            a0.wait()
            a1.wait()
            a2.wait()
            pltpu.sync_copy(rk0, ok_hbm.at[pl.ds(off, CH)])
            pltpu.sync_copy(rq0, oq_hbm.at[pl.ds(off, CH)])
            pltpu.sync_copy(rv0, ov_hbm.at[pl.ds(off, CH)])
            b0.wait()
            b1.wait()
            b2.wait()
            pltpu.sync_copy(rk1, ok_hbm.at[pl.ds(off + CH, CH)])
            pltpu.sync_copy(rq1, oq_hbm.at[pl.ds(off + CH, CH)])
            pltpu.sync_copy(rv1, ov_hbm.at[pl.ds(off + CH, CH)])

    return gather3


_GATHER3 = _make_gather3()


_QR = 8192
_ACC_R = _QR + 16
_CH = 128
_DR = NHALF // 16


def _make_scatter3():
    n1 = E_RAW // NS // _CH
    n2 = NHALF // NS // _CH
    ZSTR = _ACC_R // NS
    OSTR = _QR // NS
    DSTR = _DR // NS

    def _chunk_off(c, s, i):
        big = c * E_RAW + s * (E_RAW // NS) + i * _CH
        lp = (2 * E_RAW + c * NHALF + s * (NHALF // NS) + (i - n1) * _CH)
        return jnp.where(i < n1, big, lp)

    n_ch = n1 + n2

    @functools.partial(
        pl.kernel,
        mesh=_SC_MESH,
        out_type=[jax.ShapeDtypeStruct((N, C), jnp.float32),
                  jax.ShapeDtypeStruct((N // 16, C), jnp.float32)],
        scratch_types=[
            pltpu.VMEM_SHARED((_ACC_R, C), jnp.float32),
            pltpu.VMEM_SHARED((_DR, C), jnp.float32),
            pltpu.VMEM((_CH,), jnp.int32),
            pltpu.VMEM((_CH,), jnp.int32),
            pltpu.VMEM((_CH,), jnp.int32),
            pltpu.VMEM((_CH, C), jnp.float32),
            pltpu.VMEM((_CH, C), jnp.float32),
            pltpu.SemaphoreType.DMA,
            pltpu.SemaphoreType.DMA,
        ],
    )
    def scat_all(exv_hbm, exs_hbm, dstl_hbm, z_hbm, agg_hbm, den_hbm,
                 acc, accd, idxr, idxA, idxB, bufA, bufB, semA, semB):
        c = lax.axis_index("c")
        s = lax.axis_index("s")

        def sweep_db(con_hbm, mkidx):
            pltpu.async_copy(con_hbm.at[pl.ds(_chunk_off(c, s, 0), _CH)],
                             bufA, semA)

            @pl.loop(0, n_ch // 2)
            def _(i):
                j0 = 2 * i
                offB = _chunk_off(c, s, j0 + 1)
                hb = pltpu.async_copy(con_hbm.at[pl.ds(offB, _CH)], bufB, semB)
                pltpu.make_async_copy(con_hbm.at[pl.ds(0, _CH)], bufA,
                                      semA).wait()
                pltpu.sync_copy(dstl_hbm.at[pl.ds(_chunk_off(c, s, j0), _CH)],
                                idxr)
                mkidx(idxA)
                pltpu.sync_copy(bufA, acc_for(mkidx).at[idxA], add=True)

                @pl.when(j0 + 2 < n_ch)
                def _():
                    pltpu.async_copy(
                        con_hbm.at[pl.ds(_chunk_off(c, s, j0 + 2), _CH)],
                        bufA, semA)

                hb.wait()
                pltpu.sync_copy(dstl_hbm.at[pl.ds(offB, _CH)], idxr)
                mkidx(idxB)
                pltpu.sync_copy(bufB, acc_for(mkidx).at[idxB], add=True)

        def mk_den(dst):
            for t in range(_CH // 16):
                dst[pl.ds(t * 16, 16)] = lax.shift_right_logical(
                    idxr[pl.ds(t * 16, 16)], 4)

        def mk_q0(dst):
            for t in range(_CH // 16):
                v = idxr[pl.ds(t * 16, 16)]
                dst[pl.ds(t * 16, 16)] = jnp.where(v < _QR, v, _QR)

        def mk_q1(dst):
            for t in range(_CH // 16):
                v = idxr[pl.ds(t * 16, 16)]
                lo = v - _QR
                dst[pl.ds(t * 16, 16)] = jnp.where(lo >= 0, lo, _QR)

        def acc_for(mkidx):
            return accd if mkidx is mk_den else acc

        pltpu.sync_copy(z_hbm.at[pl.ds(0, DSTR)], accd.at[pl.ds(s * DSTR, DSTR)])
        plsc.subcore_barrier()
        sweep_db(exs_hbm, mk_den)
        plsc.subcore_barrier()
        pltpu.sync_copy(accd.at[pl.ds(s * DSTR, DSTR)],
                        den_hbm.at[pl.ds(c * _DR + s * DSTR, DSTR)])
        plsc.subcore_barrier()

        for mkidx, q in ((mk_q0, 0), (mk_q1, 1)):
            pltpu.sync_copy(z_hbm, acc.at[pl.ds(s * ZSTR, ZSTR)])
            plsc.subcore_barrier()
            sweep_db(exv_hbm, mkidx)
            plsc.subcore_barrier()
            pltpu.sync_copy(
                acc.at[pl.ds(s * OSTR, OSTR)],
                agg_hbm.at[pl.ds(c * NHALF + q * _QR + s * OSTR, OSTR)])
            plsc.subcore_barrier()

    return scat_all


_SCAT_ALL = _make_scatter3()


def _lnk(x, g, b):
    m = x.mean(-1, keepdims=True)
    v = ((x - m) ** 2).mean(-1, keepdims=True)
    return (x - m) / jnp.sqrt(v + 1e-5) * g + b


def _headmat():
    ch = lax.broadcasted_iota(jnp.int32, (C, NUM_HEADS), 0) // DH
    hh = lax.broadcasted_iota(jnp.int32, (C, NUM_HEADS), 1)
    return (ch == hh).astype(jnp.float32)


def _embed_body(ego_ref, neb_ref, scl_ref, wdt_ref, bd_ref, out_ref):
    i = pl.program_id(0)
    g = i // (NHALF // _BR)
    blk = i % (NHALF // _BR)
    scl = scl_ref[...]
    s0 = jnp.where(g == 0, scl[0, 0], scl[0, 2])
    s1 = jnp.where(g == 0, scl[0, 1], scl[0, 3])
    hw = (blk * _BR + lax.broadcasted_iota(jnp.int32, (_BR, 1), 0)).astype(jnp.float32)
    ii = jnp.floor(hw / W) - (H - 1) / 2.0
    jj = jnp.mod(hw, W) - (W - 1) / 2.0
    d = jnp.sqrt(jnp.square(s0 * ii) + jnp.square(s1 * jj))
    cc = lax.broadcasted_iota(jnp.int32, (1, C), 1)
    ce = ((cc // 2) * 2).astype(jnp.float32)
    div = jnp.exp(-ce * (math.log(10000.0) / C))
    arg = d * div
    pe = jnp.where((cc % 2) == 0, jnp.sin(arg), jnp.cos(arg)) / math.sqrt(C)
    x = jnp.where(g == 0, ego_ref[...], neb_ref[...]).T
    out_ref[...] = x + jax.lax.dot(pe, wdt_ref[...]) + bd_ref[...]


def _embed(ego2, neb2, scl, wdt, bd):
    nb = NHALF // _BR
    return pl.pallas_call(
        _embed_body,
        grid=(2 * nb,),
        in_specs=[
            pl.BlockSpec((C, _BR), lambda i: (0, i % (NHALF // _BR))),
            pl.BlockSpec((C, _BR), lambda i: (0, i % (NHALF // _BR))),
            pl.BlockSpec((1, 8), lambda i: (0, 0)),
            pl.BlockSpec((C, C), lambda i: (0, 0)),
            pl.BlockSpec((1, C), lambda i: (0, 0)),
        ],
        out_specs=pl.BlockSpec((_BR, C), lambda i: (i, 0)),
        out_shape=jax.ShapeDtypeStruct((N, C), jnp.float32),
    )(ego2, neb2, scl, wdt, bd)


def _einit_body(ev_ref, scl_ref, wd1_ref, bd1_ref, wd2_ref, bd2_ref,
                wee_ref, bee_ref, out_ref):
    i = pl.program_id(0)
    nb = E_RAW // _BR
    ev = ev_ref[...]
    scl = scl_ref[...]
    neb_area = scl[0, 0]
    ego_area = scl[0, 1]
    dis = ev[:, 0:1]
    t = jax.lax.dot(dis, wd1_ref[...]) + bd1_ref[...]
    delta = jax.lax.dot(t, wd2_ref[...]) + bd2_ref[...]
    delta = delta[:, 0]
    ddd = delta / (ev[:, 0] + 1e-7)
    v0 = (ev[:, 0] + delta) / SEARCH_RANGE
    ddn = delta ** 2 / neb_area
    v1 = (ev[:, 1] + ddn) / (1.0 + ddn)
    ddn2 = delta ** 2 / ego_area
    v1n = (ev[:, 1] * (neb_area / ego_area) + ddn2) / (1.0 + ddn2)
    v2 = (ev[:, 2] + ddd) / (1.0 + ddd)
    v3 = (ev[:, 3] + ddd) / (1.0 + ddd)
    r1 = jnp.stack([v0, v1, v2, v3], axis=-1)
    r2 = jnp.stack([v0, v1n, v2, -v3], axis=-1)
    ones = jnp.ones((_BR,), jnp.float32)
    zer = jnp.zeros((_BR,), jnp.float32)
    rc = jnp.stack([zer, ones, zer, ones], axis=-1)
    vals = jnp.where(i < nb, r1, jnp.where(i < 2 * nb, r2, rc))
    out_ref[...] = jax.lax.dot(vals, wee_ref[...]) + bee_ref[...]


def _einit(edge_vals, scl, p):
    nb = E_RAW // _BR
    return pl.pallas_call(
        _einit_body,
        grid=(E_TOT // _BR,),
        in_specs=[
            pl.BlockSpec((_BR, 4), lambda i: (i % (E_RAW // _BR), 0)),
            pl.BlockSpec((1, 8), lambda i: (0, 0)),
            pl.BlockSpec((1, 8), lambda i: (0, 0)),
            pl.BlockSpec((1, 8), lambda i: (0, 0)),
            pl.BlockSpec((8, 1), lambda i: (0, 0)),
            pl.BlockSpec((1, 1), lambda i: (0, 0)),
            pl.BlockSpec((4, C), lambda i: (0, 0)),
            pl.BlockSpec((1, C), lambda i: (0, 0)),
        ],
        out_specs=pl.BlockSpec((_BR, C), lambda i: (i, 0)),
        out_shape=jax.ShapeDtypeStruct((E_TOT, C), jnp.float32),
    )(edge_vals, scl, p['Wd1'], p['bd1'].reshape(1, 8), p['Wd2'],
      p['bd2'].reshape(1, 1), p['W_ee'], p['b_ee'].reshape(1, C))


def _qkv_body(h_ref, wq_ref, wk_ref, wv_ref, q_ref, k_ref, v_ref):
    hb = h_ref[...]
    q_ref[...] = jax.lax.dot(hb, wq_ref[...])
    k_ref[...] = jax.lax.dot(hb, wk_ref[...])
    v_ref[...] = jax.lax.dot(hb, wv_ref[...])


def _qkv(h, p):
    return pl.pallas_call(
        _qkv_body,
        grid=(N // _BR,),
        in_specs=[pl.BlockSpec((_BR, C), lambda i: (i, 0))] +
                 [pl.BlockSpec((C, C), lambda i: (0, 0))] * 3,
        out_specs=[pl.BlockSpec((_BR, C), lambda i: (i, 0))] * 3,
        out_shape=[jax.ShapeDtypeStruct((N, C), jnp.float32)] * 3,
    )(h, p['Wq'], p['Wk'], p['Wv'])


def _escore_body(e_ref, we_ref, ks_ref, qd_ref, vs_ref, dl_ref,
                 eout_ref, exv_ref, exs_ref):
    pe = jax.lax.dot(e_ref[...], we_ref[...])
    s = ks_ref[...] * qd_ref[...] * pe * (1.0 / math.sqrt(DH))
    eout_ref[...] = s
    hm = _headmat()
    logits = jnp.clip(jax.lax.dot(s, hm, precision=jax.lax.Precision.HIGHEST),
                      -5.0, 5.0)
    ex = jnp.exp(logits)
    exf = jax.lax.dot(ex, hm.T, precision=jax.lax.Precision.HIGHEST)
    exv_ref[...] = exf * vs_ref[...]
    sh = (jnp.mod(dl_ref[...], 16) ==
          lax.broadcasted_iota(jnp.int32, (1, 16), 1)).astype(jnp.float32)
    cc = lax.broadcasted_iota(jnp.int32, (16, C), 1)
    rr = lax.broadcasted_iota(jnp.int32, (16, C), 0)
    sp = (cc // 8 == rr).astype(jnp.float32)
    cc8 = lax.broadcasted_iota(jnp.int32, (NUM_HEADS, C), 1)
    rr8 = lax.broadcasted_iota(jnp.int32, (NUM_HEADS, C), 0)
    hp = (jnp.mod(cc8, 8) == rr8).astype(jnp.float32)
    exs_ref[...] = (jax.lax.dot(sh, sp, precision=jax.lax.Precision.HIGHEST) *
                    jax.lax.dot(ex, hp, precision=jax.lax.Precision.HIGHEST))


def _escore(e, ksrc, qdst, vsrc, dl2, p):
    return pl.pallas_call(
        _escore_body,
        grid=(E_TOT // _BR,),
        in_specs=[
            pl.BlockSpec((_BR, C), lambda i: (i, 0)),
            pl.BlockSpec((C, C), lambda i: (0, 0)),
            pl.BlockSpec((_BR, C), lambda i: (i, 0)),
            pl.BlockSpec((_BR, C), lambda i: (i, 0)),
            pl.BlockSpec((_BR, C), lambda i: (i, 0)),
            pl.BlockSpec((_BR, 1), lambda i: (i, 0)),
        ],
        out_specs=[pl.BlockSpec((_BR, C), lambda i: (i, 0))] * 3,
        out_shape=[jax.ShapeDtypeStruct((E_TOT, C), jnp.float32)] * 3,
    )(e, p['We'], ksrc, qdst, vsrc, dl2)


def _hupd_body(h_ref, agg_ref, den_ref, wo_ref, bo_ref, g1_ref, b1_ref,
               wf1_ref, bf1_ref, wf2_ref, bf2_ref, g2_ref, b2_ref, out_ref):
    den8 = den_ref[...] + 1e-9
    den = jax.lax.dot(den8, _headmat().T,
                      precision=jax.lax.Precision.HIGHEST)
    hagg = agg_ref[...] / den
    h = h_ref[...]
    h1 = _lnk(h + jax.lax.dot(hagg, wo_ref[...]) + bo_ref[...],
              g1_ref[...], b1_ref[...])
    ff = jnp.maximum(jax.lax.dot(h1, wf1_ref[...]) + bf1_ref[...], 0.0)
    out_ref[...] = _lnk(h1 + jax.lax.dot(ff, wf2_ref[...]) + bf2_ref[...],
                        g2_ref[...], b2_ref[...])


def _hupd(h, agg, denp, p):
    r1 = lambda a: a.reshape(1, -1)
    return pl.pallas_call(
        _hupd_body,
        grid=(N // _BR,),
        in_specs=[
            pl.BlockSpec((_BR, C), lambda i: (i, 0)),
            pl.BlockSpec((_BR, C), lambda i: (i, 0)),
            pl.BlockSpec((_BR, NUM_HEADS), lambda i: (i, 0)),
            pl.BlockSpec((C, C), lambda i: (0, 0)),
            pl.BlockSpec((1, C), lambda i: (0, 0)),
            pl.BlockSpec((1, C), lambda i: (0, 0)),
            pl.BlockSpec((1, C), lambda i: (0, 0)),
            pl.BlockSpec((C, 2 * C), lambda i: (0, 0)),
            pl.BlockSpec((1, 2 * C), lambda i: (0, 0)),
            pl.BlockSpec((2 * C, C), lambda i: (0, 0)),
            pl.BlockSpec((1, C), lambda i: (0, 0)),
            pl.BlockSpec((1, C), lambda i: (0, 0)),
            pl.BlockSpec((1, C), lambda i: (0, 0)),
        ],
        out_specs=pl.BlockSpec((_BR, C), lambda i: (i, 0)),
        out_shape=jax.ShapeDtypeStruct((N, C), jnp.float32),
    )(h, agg, denp, p['Wo'], r1(p['bo']), r1(p['ln1g']), r1(p['ln1b']),
      p['Wf1'], r1(p['bf1']), p['Wf2'], r1(p['bf2']),
      r1(p['ln2g']), r1(p['ln2b']))


def _eupd_body(e_ref, eo_ref, wo_ref, bo_ref, g1_ref, b1_ref,
               wf1_ref, bf1_ref, wf2_ref, bf2_ref, g2_ref, b2_ref, out_ref):
    e1 = _lnk(e_ref[...] + jax.lax.dot(eo_ref[...], wo_ref[...]) + bo_ref[...],
              g1_ref[...], b1_ref[...])
    ff = jnp.maximum(jax.lax.dot(e1, wf1_ref[...]) + bf1_ref[...], 0.0)
    out_ref[...] = _lnk(e1 + jax.lax.dot(ff, wf2_ref[...]) + bf2_ref[...],
                        g2_ref[...], b2_ref[...])


def _eupd(e, eout, p):
    r1 = lambda a: a.reshape(1, -1)
    return pl.pallas_call(
        _eupd_body,
        grid=(E_TOT // _BR,),
        in_specs=[
            pl.BlockSpec((_BR, C), lambda i: (i, 0)),
            pl.BlockSpec((_BR, C), lambda i: (i, 0)),
            pl.BlockSpec((C, C), lambda i: (0, 0)),
            pl.BlockSpec((1, C), lambda i: (0, 0)),
            pl.BlockSpec((1, C), lambda i: (0, 0)),
            pl.BlockSpec((1, C), lambda i: (0, 0)),
            pl.BlockSpec((C, 2 * C), lambda i: (0, 0)),
            pl.BlockSpec((1, 2 * C), lambda i: (0, 0)),
            pl.BlockSpec((2 * C, C), lambda i: (0, 0)),
            pl.BlockSpec((1, C), lambda i: (0, 0)),
            pl.BlockSpec((1, C), lambda i: (0, 0)),
            pl.BlockSpec((1, C), lambda i: (0, 0)),
        ],
        out_specs=pl.BlockSpec((_BR, C), lambda i: (i, 0)),
        out_shape=jax.ShapeDtypeStruct((E_TOT, C), jnp.float32),
    )(e, eout, p['WoE'], r1(p['boE']), r1(p['lnE1g']), r1(p['lnE1b']),
      p['WfE1'], r1(p['bfE1']), p['WfE2'], r1(p['bfE2']),
      r1(p['lnE2g']), r1(p['lnE2b']))


def _out_transpose_body(h_ref, o_ref):
    o_ref[...] = h_ref[...].reshape(8, W, C).transpose(2, 0, 1)


def _gt_layer(h, e, src_g, dst_g, dst_l, dl2, zrows, p):
    q, k, v = _qkv(h, p)
    ksrc, qdst, vsrc = _GATHER3(k, q, v, src_g, dst_g)
    e_out, exv, exs = _escore(e, ksrc, qdst, vsrc, dl2, p)
    agg, denp = _SCAT_ALL(exv, exs, dst_l, zrows)
    e2 = _eupd(e, e_out, p)
    h2 = _hupd(h, agg, denp.reshape(N, NUM_HEADS), p)
    return h2, e2


def kernel(ego, neb, neb_confidence_map, neb_point_cloud_range, edge_index, edge_vals, params):
    neb_pcr = neb_point_cloud_range
    s0e = (PCR[4] - PCR[1]) / H
    s1e = (PCR[3] - PCR[0]) / W
    s0n = (neb_pcr[4] - neb_pcr[1]) / H
    s1n = (neb_pcr[3] - neb_pcr[0]) / W
    neb_area = s0n * s1n
    ego_area = jnp.float32(s0e * s1e)
    scl_embed = jnp.stack([jnp.float32(s0e), jnp.float32(s1e), s0n, s1n,
                           jnp.float32(0), jnp.float32(0), jnp.float32(0),
                           jnp.float32(0)]).reshape(1, 8).astype(jnp.float32)
    scl_einit = jnp.stack([neb_area, ego_area, jnp.float32(0), jnp.float32(0),
                           jnp.float32(0), jnp.float32(0), jnp.float32(0),
                           jnp.float32(0)]).reshape(1, 8).astype(jnp.float32)

    ei0 = edge_index[0].astype(jnp.int32)
    ei1 = edge_index[1].astype(jnp.int32)
    loop = jnp.arange(N, dtype=jnp.int32)
    ar = jnp.arange(NHALF, dtype=jnp.int32)
    src_g = jnp.concatenate([ei0 + NHALF, ei1, loop])
    dst_g = jnp.concatenate([ei1, ei0 + NHALF, loop])
    dst_l = jnp.concatenate([ei1, ei0, ar, ar])
    dl2 = dst_l.reshape(E_TOT, 1)
    zrows = jnp.zeros((_ACC_R // NS, C), jnp.float32)

    h = _embed(ego.reshape(C, NHALF), neb.reshape(C, NHALF), scl_embed,
               params['W_dis'].T, params['b_dis'].reshape(1, C))
    e = _einit(edge_vals, scl_einit, params)
    for l in range(N_LAYERS):
        h, e = _gt_layer(h, e, src_g, dst_g, dst_l, dl2, zrows,
                         params['layers'][l])

    out = pl.pallas_call(
        _out_transpose_body,
        grid=(H // 8,),
        in_specs=[pl.BlockSpec((8 * W, C), lambda i: (i, 0))],
        out_specs=pl.BlockSpec((C, 8, W), lambda i: (0, i, 0)),
        out_shape=jax.ShapeDtypeStruct((C, H, W), jnp.float32),
    )(h[:NHALF])
    return out


# speedup vs baseline: 1.2786x; 1.1444x over previous
import functools
import math

import jax
import jax.numpy as jnp
from jax import lax
from jax.experimental import pallas as pl
from jax.experimental.pallas import tpu as pltpu
from jax.experimental.pallas import tpu_sc as plsc

C = 128
H = 128
W = 128
E_RAW = 131072
NHALF = H * W
N = 2 * H * W
NUM_HEADS = 8
DH = C // NUM_HEADS
N_LAYERS = 2
SEARCH_RANGE = 3.0
PCR = (-140.8, -40.0, -3.0, 140.8, 40.0, 1.0)

NC = 2
NS = 16
NW = NC * NS
E_TOT = 2 * E_RAW + N

_SC_MESH = plsc.VectorSubcoreMesh(core_axis_name="c", subcore_axis_name="s")
_BR = 1024


def _make_gather3():
    per_w = E_TOT // NW
    CH = 128
    n_ch = per_w // CH

    @functools.partial(
        pl.kernel,
        mesh=_SC_MESH,
        out_type=[jax.ShapeDtypeStruct((E_TOT, C), jnp.float32)] * 3,
        scratch_types=[
            pltpu.VMEM((2 * CH,), jnp.int32),
            pltpu.VMEM((2 * CH,), jnp.int32),
            pltpu.VMEM((CH, C), jnp.float32),
            pltpu.VMEM((CH, C), jnp.float32),
            pltpu.VMEM((CH, C), jnp.float32),
            pltpu.VMEM((CH, C), jnp.float32),
            pltpu.VMEM((CH, C), jnp.float32),
            pltpu.VMEM((CH, C), jnp.float32),
            pltpu.SemaphoreType.DMA,
            pltpu.SemaphoreType.DMA,
        ],
    )
    def gather3(k_hbm, q_hbm, v_hbm, src_hbm, dst_hbm,
                ok_hbm, oq_hbm, ov_hbm, idxs, idxd,
                rk0, rq0, rv0, rk1, rq1, rv1, semA, semB):
        wid = lax.axis_index("s") * NC + lax.axis_index("c")
        base = wid * per_w

        @pl.loop(0, n_ch // 2)
        def _(j):
            off = base + 2 * j * CH
            pltpu.sync_copy(src_hbm.at[pl.ds(off, 2 * CH)], idxs)
            pltpu.sync_copy(dst_hbm.at[pl.ds(off, 2 * CH)], idxd)
            sA, dA = idxs.at[pl.ds(0, CH)], idxd.at[pl.ds(0, CH)]
            sB, dB = idxs.at[pl.ds(CH, CH)], idxd.at[pl.ds(CH, CH)]
            a0 = pltpu.async_copy(k_hbm.at[sA], rk0, semA)
            a1 = pltpu.async_copy(q_hbm.at[dA], rq0, semA)
            a2 = pltpu.async_copy(v_hbm.at[sA], rv0, semA)
            b0 = pltpu.async_copy(k_hbm.at[sB], rk1, semB)
            b1 = pltpu.async_copy(q_hbm.at[dB], rq1, semB)
            b2 = pltpu.async_copy(v_hbm.at[sB], rv1, semB)
            a0.wait()
            a1.wait()
            a2.wait()
            pltpu.sync_copy(rk0, ok_hbm.at[pl.ds(off, CH)])
            pltpu.sync_copy(rq0, oq_hbm.at[pl.ds(off, CH)])
            pltpu.sync_copy(rv0, ov_hbm.at[pl.ds(off, CH)])
            b0.wait()
            b1.wait()
            b2.wait()
            pltpu.sync_copy(rk1, ok_hbm.at[pl.ds(off + CH, CH)])
            pltpu.sync_copy(rq1, oq_hbm.at[pl.ds(off + CH, CH)])
            pltpu.sync_copy(rv1, ov_hbm.at[pl.ds(off + CH, CH)])

    return gather3


_GATHER3 = _make_gather3()


_QR = 8192
_ACC_R = _QR + 16
_CH = 128
_DR = NHALF // 16


def _make_scatter3():
    n1 = E_RAW // NS // _CH
    n2 = NHALF // NS // _CH
    ZSTR = _ACC_R // NS
    OSTR = _QR // NS
    DSTR = _DR // NS

    def _chunk_off(c, s, i):
        big = c * E_RAW + s * (E_RAW // NS) + i * _CH
        lp = (2 * E_RAW + c * NHALF + s * (NHALF // NS) + (i - n1) * _CH)
        return jnp.where(i < n1, big, lp)

    n_ch = n1 + n2

    @functools.partial(
        pl.kernel,
        mesh=_SC_MESH,
        out_type=[jax.ShapeDtypeStruct((N, C), jnp.float32),
                  jax.ShapeDtypeStruct((N // 16, C), jnp.float32)],
        scratch_types=[
            pltpu.VMEM_SHARED((_ACC_R, C), jnp.float32),
            pltpu.VMEM_SHARED((_DR, C), jnp.float32),
            pltpu.VMEM((_CH,), jnp.int32),
            pltpu.VMEM((_CH,), jnp.int32),
            pltpu.VMEM((_CH,), jnp.int32),
            pltpu.VMEM((_CH, C), jnp.float32),
            pltpu.VMEM((_CH, C), jnp.float32),
            pltpu.SemaphoreType.DMA,
            pltpu.SemaphoreType.DMA,
        ],
    )
    def scat_all(exv_hbm, exs_hbm, dstl_hbm, z_hbm, agg_hbm, den_hbm,
                 acc, accd, idxr, idxA, idxB, bufA, bufB, semA, semB):
        c = lax.axis_index("c")
        s = lax.axis_index("s")

        def sweep_db(con_hbm, mkidx):
            pltpu.async_copy(con_hbm.at[pl.ds(_chunk_off(c, s, 0), _CH)],
                             bufA, semA)

            @pl.loop(0, n_ch // 2)
            def _(i):
                j0 = 2 * i
                offB = _chunk_off(c, s, j0 + 1)
                hb = pltpu.async_copy(con_hbm.at[pl.ds(offB, _CH)], bufB, semB)
                pltpu.make_async_copy(con_hbm.at[pl.ds(0, _CH)], bufA,
                                      semA).wait()
                pltpu.sync_copy(dstl_hbm.at[pl.ds(_chunk_off(c, s, j0), _CH)],
                                idxr)
                mkidx(idxA)
                pltpu.sync_copy(bufA, acc_for(mkidx).at[idxA], add=True)

                @pl.when(j0 + 2 < n_ch)
                def _():
                    pltpu.async_copy(
                        con_hbm.at[pl.ds(_chunk_off(c, s, j0 + 2), _CH)],
                        bufA, semA)

                hb.wait()
                pltpu.sync_copy(dstl_hbm.at[pl.ds(offB, _CH)], idxr)
                mkidx(idxB)
                pltpu.sync_copy(bufB, acc_for(mkidx).at[idxB], add=True)

        def mk_den(dst):
            for t in range(_CH // 16):
                dst[pl.ds(t * 16, 16)] = lax.shift_right_logical(
                    idxr[pl.ds(t * 16, 16)], 4)

        def mk_q0(dst):
            for t in range(_CH // 16):
                v = idxr[pl.ds(t * 16, 16)]
                dst[pl.ds(t * 16, 16)] = jnp.where(v < _QR, v, _QR)

        def mk_q1(dst):
            for t in range(_CH // 16):
                v = idxr[pl.ds(t * 16, 16)]
                lo = v - _QR
                dst[pl.ds(t * 16, 16)] = jnp.where(lo >= 0, lo, _QR)

        def acc_for(mkidx):
            return accd if mkidx is mk_den else acc

        pltpu.sync_copy(z_hbm.at[pl.ds(0, DSTR)], accd.at[pl.ds(s * DSTR, DSTR)])
        plsc.subcore_barrier()
        sweep_db(exs_hbm, mk_den)
        plsc.subcore_barrier()
        pltpu.sync_copy(accd.at[pl.ds(s * DSTR, DSTR)],
                        den_hbm.at[pl.ds(c * _DR + s * DSTR, DSTR)])
        plsc.subcore_barrier()

        for mkidx, q in ((mk_q0, 0), (mk_q1, 1)):
            pltpu.sync_copy(z_hbm, acc.at[pl.ds(s * ZSTR, ZSTR)])
            plsc.subcore_barrier()
            sweep_db(exv_hbm, mkidx)
            plsc.subcore_barrier()
            pltpu.sync_copy(
                acc.at[pl.ds(s * OSTR, OSTR)],
                agg_hbm.at[pl.ds(c * NHALF + q * _QR + s * OSTR, OSTR)])
            plsc.subcore_barrier()

    return scat_all


_SCAT_ALL = _make_scatter3()


def _lnk(x, g, b):
    m = x.mean(-1, keepdims=True)
    v = ((x - m) ** 2).mean(-1, keepdims=True)
    return (x - m) / jnp.sqrt(v + 1e-5) * g + b


def _headmat():
    ch = lax.broadcasted_iota(jnp.int32, (C, NUM_HEADS), 0) // DH
    hh = lax.broadcasted_iota(jnp.int32, (C, NUM_HEADS), 1)
    return (ch == hh).astype(jnp.float32)


def _embed_body(ego_ref, neb_ref, scl_ref, wdt_ref, bd_ref, out_ref):
    i = pl.program_id(0)
    g = i // (NHALF // _BR)
    blk = i % (NHALF // _BR)
    scl = scl_ref[...]
    s0 = jnp.where(g == 0, scl[0, 0], scl[0, 2])
    s1 = jnp.where(g == 0, scl[0, 1], scl[0, 3])
    hw = (blk * _BR + lax.broadcasted_iota(jnp.int32, (_BR, 1), 0)).astype(jnp.float32)
    ii = jnp.floor(hw / W) - (H - 1) / 2.0
    jj = jnp.mod(hw, W) - (W - 1) / 2.0
    d = jnp.sqrt(jnp.square(s0 * ii) + jnp.square(s1 * jj))
    cc = lax.broadcasted_iota(jnp.int32, (1, C), 1)
    ce = ((cc // 2) * 2).astype(jnp.float32)
    div = jnp.exp(-ce * (math.log(10000.0) / C))
    arg = d * div
    pe = jnp.where((cc % 2) == 0, jnp.sin(arg), jnp.cos(arg)) / math.sqrt(C)
    x = jnp.where(g == 0, ego_ref[...], neb_ref[...]).T
    out_ref[...] = x + jax.lax.dot(pe, wdt_ref[...]) + bd_ref[...]


def _embed(ego2, neb2, scl, wdt, bd):
    nb = NHALF // _BR
    return pl.pallas_call(
        _embed_body,
        grid=(2 * nb,),
        in_specs=[
            pl.BlockSpec((C, _BR), lambda i: (0, i % (NHALF // _BR))),
            pl.BlockSpec((C, _BR), lambda i: (0, i % (NHALF // _BR))),
            pl.BlockSpec((1, 8), lambda i: (0, 0)),
            pl.BlockSpec((C, C), lambda i: (0, 0)),
            pl.BlockSpec((1, C), lambda i: (0, 0)),
        ],
        out_specs=pl.BlockSpec((_BR, C), lambda i: (i, 0)),
        out_shape=jax.ShapeDtypeStruct((N, C), jnp.float32),
    )(ego2, neb2, scl, wdt, bd)


def _einit_body(ev_ref, scl_ref, wd1_ref, bd1_ref, wd2_ref, bd2_ref,
                wee_ref, bee_ref, out_ref):
    i = pl.program_id(0)
    nb = E_RAW // _BR
    ev = ev_ref[...]
    scl = scl_ref[...]
    neb_area = scl[0, 0]
    ego_area = scl[0, 1]
    dis = ev[:, 0:1]
    t = jax.lax.dot(dis, wd1_ref[...]) + bd1_ref[...]
    delta = jax.lax.dot(t, wd2_ref[...]) + bd2_ref[...]
    delta = delta[:, 0]
    ddd = delta / (ev[:, 0] + 1e-7)
    v0 = (ev[:, 0] + delta) / SEARCH_RANGE
    ddn = delta ** 2 / neb_area
    v1 = (ev[:, 1] + ddn) / (1.0 + ddn)
    ddn2 = delta ** 2 / ego_area
    v1n = (ev[:, 1] * (neb_area / ego_area) + ddn2) / (1.0 + ddn2)
    v2 = (ev[:, 2] + ddd) / (1.0 + ddd)
    v3 = (ev[:, 3] + ddd) / (1.0 + ddd)
    r1 = jnp.stack([v0, v1, v2, v3], axis=-1)
    r2 = jnp.stack([v0, v1n, v2, -v3], axis=-1)
    ones = jnp.ones((_BR,), jnp.float32)
    zer = jnp.zeros((_BR,), jnp.float32)
    rc = jnp.stack([zer, ones, zer, ones], axis=-1)
    vals = jnp.where(i < nb, r1, jnp.where(i < 2 * nb, r2, rc))
    out_ref[...] = jax.lax.dot(vals, wee_ref[...]) + bee_ref[...]


def _einit(edge_vals, scl, p):
    nb = E_RAW // _BR
    return pl.pallas_call(
        _einit_body,
        grid=(E_TOT // _BR,),
        in_specs=[
            pl.BlockSpec((_BR, 4), lambda i: (i % (E_RAW // _BR), 0)),
            pl.BlockSpec((1, 8), lambda i: (0, 0)),
            pl.BlockSpec((1, 8), lambda i: (0, 0)),
            pl.BlockSpec((1, 8), lambda i: (0, 0)),
            pl.BlockSpec((8, 1), lambda i: (0, 0)),
            pl.BlockSpec((1, 1), lambda i: (0, 0)),
            pl.BlockSpec((4, C), lambda i: (0, 0)),
            pl.BlockSpec((1, C), lambda i: (0, 0)),
        ],
        out_specs=pl.BlockSpec((_BR, C), lambda i: (i, 0)),
        out_shape=jax.ShapeDtypeStruct((E_TOT, C), jnp.float32),
    )(edge_vals, scl, p['Wd1'], p['bd1'].reshape(1, 8), p['Wd2'],
      p['bd2'].reshape(1, 1), p['W_ee'], p['b_ee'].reshape(1, C))


def _qkv_body(h_ref, wq_ref, wk_ref, wv_ref, q_ref, k_ref, v_ref):
    hb = h_ref[...]
    q_ref[...] = jax.lax.dot(hb, wq_ref[...])
    k_ref[...] = jax.lax.dot(hb, wk_ref[...])
    v_ref[...] = jax.lax.dot(hb, wv_ref[...])


def _qkv(h, p):
    return pl.pallas_call(
        _qkv_body,
        grid=(N // _BR,),
        in_specs=[pl.BlockSpec((_BR, C), lambda i: (i, 0))] +
                 [pl.BlockSpec((C, C), lambda i: (0, 0))] * 3,
        out_specs=[pl.BlockSpec((_BR, C), lambda i: (i, 0))] * 3,
        out_shape=[jax.ShapeDtypeStruct((N, C), jnp.float32)] * 3,
    )(h, p['Wq'], p['Wk'], p['Wv'])


def _escore_body(e_ref, we_ref, ks_ref, qd_ref, vs_ref, dl_ref,
                 eout_ref, exv_ref, exs_ref):
    pe = jax.lax.dot(e_ref[...], we_ref[...])
    s = ks_ref[...] * qd_ref[...] * pe * (1.0 / math.sqrt(DH))
    eout_ref[...] = s
    hm = _headmat()
    logits = jnp.clip(jax.lax.dot(s, hm, precision=jax.lax.Precision.HIGHEST),
                      -5.0, 5.0)
    ex = jnp.exp(logits)
    exf = jax.lax.dot(ex, hm.T, precision=jax.lax.Precision.HIGHEST)
    exv_ref[...] = exf * vs_ref[...]
    sh = (jnp.mod(dl_ref[...], 16) ==
          lax.broadcasted_iota(jnp.int32, (1, 16), 1)).astype(jnp.float32)
    cc = lax.broadcasted_iota(jnp.int32, (16, C), 1)
    rr = lax.broadcasted_iota(jnp.int32, (16, C), 0)
    sp = (cc // 8 == rr).astype(jnp.float32)
    cc8 = lax.broadcasted_iota(jnp.int32, (NUM_HEADS, C), 1)
    rr8 = lax.broadcasted_iota(jnp.int32, (NUM_HEADS, C), 0)
    hp = (jnp.mod(cc8, 8) == rr8).astype(jnp.float32)
    exs_ref[...] = (jax.lax.dot(sh, sp, precision=jax.lax.Precision.HIGHEST) *
                    jax.lax.dot(ex, hp, precision=jax.lax.Precision.HIGHEST))


def _escore(e, ksrc, qdst, vsrc, dl2, p):
    return pl.pallas_call(
        _escore_body,
        grid=(E_TOT // _BR,),
        in_specs=[
            pl.BlockSpec((_BR, C), lambda i: (i, 0)),
            pl.BlockSpec((C, C), lambda i: (0, 0)),
            pl.BlockSpec((_BR, C), lambda i: (i, 0)),
            pl.BlockSpec((_BR, C), lambda i: (i, 0)),
            pl.BlockSpec((_BR, C), lambda i: (i, 0)),
            pl.BlockSpec((_BR, 1), lambda i: (i, 0)),
        ],
        out_specs=[pl.BlockSpec((_BR, C), lambda i: (i, 0))] * 3,
        out_shape=[jax.ShapeDtypeStruct((E_TOT, C), jnp.float32)] * 3,
    )(e, p['We'], ksrc, qdst, vsrc, dl2)


def _hupd_body(h_ref, agg_ref, den_ref, wo_ref, bo_ref, g1_ref, b1_ref,
               wf1_ref, bf1_ref, wf2_ref, bf2_ref, g2_ref, b2_ref, out_ref):
    den8 = den_ref[...] + 1e-9
    den = jax.lax.dot(den8, _headmat().T,
                      precision=jax.lax.Precision.HIGHEST)
    hagg = agg_ref[...] / den
    h = h_ref[...]
    h1 = _lnk(h + jax.lax.dot(hagg, wo_ref[...]) + bo_ref[...],
              g1_ref[...], b1_ref[...])
    ff = jnp.maximum(jax.lax.dot(h1, wf1_ref[...]) + bf1_ref[...], 0.0)
    out_ref[...] = _lnk(h1 + jax.lax.dot(ff, wf2_ref[...]) + bf2_ref[...],
                        g2_ref[...], b2_ref[...])


def _hupd(h, agg, denp, p):
    r1 = lambda a: a.reshape(1, -1)
    return pl.pallas_call(
        _hupd_body,
        grid=(N // _BR,),
        in_specs=[
            pl.BlockSpec((_BR, C), lambda i: (i, 0)),
            pl.BlockSpec((_BR, C), lambda i: (i, 0)),
            pl.BlockSpec((_BR, NUM_HEADS), lambda i: (i, 0)),
            pl.BlockSpec((C, C), lambda i: (0, 0)),
            pl.BlockSpec((1, C), lambda i: (0, 0)),
            pl.BlockSpec((1, C), lambda i: (0, 0)),
            pl.BlockSpec((1, C), lambda i: (0, 0)),
            pl.BlockSpec((C, 2 * C), lambda i: (0, 0)),
            pl.BlockSpec((1, 2 * C), lambda i: (0, 0)),
            pl.BlockSpec((2 * C, C), lambda i: (0, 0)),
            pl.BlockSpec((1, C), lambda i: (0, 0)),
            pl.BlockSpec((1, C), lambda i: (0, 0)),
            pl.BlockSpec((1, C), lambda i: (0, 0)),
        ],
        out_specs=pl.BlockSpec((_BR, C), lambda i: (i, 0)),
        out_shape=jax.ShapeDtypeStruct((N, C), jnp.float32),
    )(h, agg, denp, p['Wo'], r1(p['bo']), r1(p['ln1g']), r1(p['ln1b']),
      p['Wf1'], r1(p['bf1']), p['Wf2'], r1(p['bf2']),
      r1(p['ln2g']), r1(p['ln2b']))


def _eupd_body(e_ref, eo_ref, wo_ref, bo_ref, g1_ref, b1_ref,
               wf1_ref, bf1_ref, wf2_ref, bf2_ref, g2_ref, b2_ref, out_ref):
    e1 = _lnk(e_ref[...] + jax.lax.dot(eo_ref[...], wo_ref[...]) + bo_ref[...],
              g1_ref[...], b1_ref[...])
    ff = jnp.maximum(jax.lax.dot(e1, wf1_ref[...]) + bf1_ref[...], 0.0)
    out_ref[...] = _lnk(e1 + jax.lax.dot(ff, wf2_ref[...]) + bf2_ref[...],
                        g2_ref[...], b2_ref[...])


def _eupd(e, eout, p):
    r1 = lambda a: a.reshape(1, -1)
    return pl.pallas_call(
        _eupd_body,
        grid=(E_TOT // _BR,),
        in_specs=[
            pl.BlockSpec((_BR, C), lambda i: (i, 0)),
            pl.BlockSpec((_BR, C), lambda i: (i, 0)),
            pl.BlockSpec((C, C), lambda i: (0, 0)),
            pl.BlockSpec((1, C), lambda i: (0, 0)),
            pl.BlockSpec((1, C), lambda i: (0, 0)),
            pl.BlockSpec((1, C), lambda i: (0, 0)),
            pl.BlockSpec((C, 2 * C), lambda i: (0, 0)),
            pl.BlockSpec((1, 2 * C), lambda i: (0, 0)),
            pl.BlockSpec((2 * C, C), lambda i: (0, 0)),
            pl.BlockSpec((1, C), lambda i: (0, 0)),
            pl.BlockSpec((1, C), lambda i: (0, 0)),
            pl.BlockSpec((1, C), lambda i: (0, 0)),
        ],
        out_specs=pl.BlockSpec((_BR, C), lambda i: (i, 0)),
        out_shape=jax.ShapeDtypeStruct((E_TOT, C), jnp.float32),
    )(e, eout, p['WoE'], r1(p['boE']), r1(p['lnE1g']), r1(p['lnE1b']),
      p['WfE1'], r1(p['bfE1']), p['WfE2'], r1(p['bfE2']),
      r1(p['lnE2g']), r1(p['lnE2b']))


def _out_transpose_body(h_ref, o_ref):
    o_ref[...] = h_ref[...].reshape(8, W, C).transpose(2, 0, 1)


def _gt_layer(h, e, src_g, dst_g, dst_l, dl2, zrows, p):
    q, k, v = _qkv(h, p)
    ksrc, qdst, vsrc = _GATHER3(k, q, v, src_g, dst_g)
    e_out, exv, exs = _escore(e, ksrc, qdst, vsrc, dl2, p)
    agg, denp = _SCAT_ALL(exv, exs, dst_l, zrows)
    e2 = _eupd(e, e_out, p)
    h2 = _hupd(h, agg, denp.reshape(N, NUM_HEADS), p)
    return h2, e2


def kernel(ego, neb, neb_confidence_map, neb_point_cloud_range, edge_index, edge_vals, params):
    neb_pcr = neb_point_cloud_range
    s0e = (PCR[4] - PCR[1]) / H
    s1e = (PCR[3] - PCR[0]) / W
    s0n = (neb_pcr[4] - neb_pcr[1]) / H
    s1n = (neb_pcr[3] - neb_pcr[0]) / W
    neb_area = s0n * s1n
    ego_area = jnp.float32(s0e * s1e)
    scl_embed = jnp.stack([jnp.float32(s0e), jnp.float32(s1e), s0n, s1n,
                           jnp.float32(0), jnp.float32(0), jnp.float32(0),
                           jnp.float32(0)]).reshape(1, 8).astype(jnp.float32)
    scl_einit = jnp.stack([neb_area, ego_area, jnp.float32(0), jnp.float32(0),
                           jnp.float32(0), jnp.float32(0), jnp.float32(0),
                           jnp.float32(0)]).reshape(1, 8).astype(jnp.float32)

    ei0 = edge_index[0].astype(jnp.int32)
    ei1 = edge_index[1].astype(jnp.int32)
    loop = jnp.arange(N, dtype=jnp.int32)
    ar = jnp.arange(NHALF, dtype=jnp.int32)
    src_g = jnp.concatenate([ei0 + NHALF, ei1, loop])
    dst_g = jnp.concatenate([ei1, ei0 + NHALF, loop])
    dst_l = jnp.concatenate([ei1, ei0, ar, ar])
    dl2 = dst_l.reshape(E_TOT, 1)
    zrows = jnp.zeros((_ACC_R // NS, C), jnp.float32)

    h = _embed(ego.reshape(C, NHALF), neb.reshape(C, NHALF), scl_embed,
               params['W_dis'].T, params['b_dis'].reshape(1, C))
    e = _einit(edge_vals, scl_einit, params)
    for l in range(N_LAYERS):
        h, e = _gt_layer(h, e, src_g, dst_g, dst_l, dl2, zrows,
                         params['layers'][l])

    out = pl.pallas_call(
        _out_transpose_body,
        grid=(H // 8,),
        in_specs=[pl.BlockSpec((8 * W, C), lambda i: (i, 0))],
        out_specs=pl.BlockSpec((C, 8, W), lambda i: (0, i, 0)),
        out_shape=jax.ShapeDtypeStruct((C, H, W), jnp.float32),
    )(h)
    return out


# speedup vs baseline: 1.3777x; 1.0775x over previous
import functools
import math

import jax
import jax.numpy as jnp
from jax import lax
from jax.experimental import pallas as pl
from jax.experimental.pallas import tpu as pltpu
from jax.experimental.pallas import tpu_sc as plsc

C = 128
H = 128
W = 128
E_RAW = 131072
NHALF = H * W
N = 2 * H * W
NUM_HEADS = 8
DH = C // NUM_HEADS
N_LAYERS = 2
SEARCH_RANGE = 3.0
PCR = (-140.8, -40.0, -3.0, 140.8, 40.0, 1.0)

NC = 2
NS = 16
NW = NC * NS
E_TOT = 2 * E_RAW + N

_SC_MESH = plsc.VectorSubcoreMesh(core_axis_name="c", subcore_axis_name="s")
_BR = 2048


def _make_gather3():
    per_w = E_TOT // NW
    CH = 128
    n_ch = per_w // CH

    @functools.partial(
        pl.kernel,
        mesh=_SC_MESH,
        out_type=[jax.ShapeDtypeStruct((E_TOT, C), jnp.float32)] * 3,
        scratch_types=[
            pltpu.VMEM((2 * CH,), jnp.int32),
            pltpu.VMEM((2 * CH,), jnp.int32),
            pltpu.VMEM((CH, C), jnp.float32),
            pltpu.VMEM((CH, C), jnp.float32),
            pltpu.VMEM((CH, C), jnp.float32),
            pltpu.VMEM((CH, C), jnp.float32),
            pltpu.VMEM((CH, C), jnp.float32),
            pltpu.VMEM((CH, C), jnp.float32),
            pltpu.SemaphoreType.DMA,
            pltpu.SemaphoreType.DMA,
        ],
    )
    def gather3(k_hbm, q_hbm, v_hbm, src_hbm, dst_hbm,
                ok_hbm, oq_hbm, ov_hbm, idxs, idxd,
                rk0, rq0, rv0, rk1, rq1, rv1, semA, semB):
        wid = lax.axis_index("s") * NC + lax.axis_index("c")
        base = wid * per_w

        @pl.loop(0, n_ch // 2)
        def _(j):
            off = base + 2 * j * CH
            pltpu.sync_copy(src_hbm.at[pl.ds(off, 2 * CH)], idxs)
            pltpu.sync_copy(dst_hbm.at[pl.ds(off, 2 * CH)], idxd)
            sA, dA = idxs.at[pl.ds(0, CH)], idxd.at[pl.ds(0, CH)]
            sB, dB = idxs.at[pl.ds(CH, CH)], idxd.at[pl.ds(CH, CH)]
            a0 = pltpu.async_copy(k_hbm.at[sA], rk0, semA)
            a1 = pltpu.async_copy(q_hbm.at[dA], rq0, semA)
            a2 = pltpu.async_copy(v_hbm.at[sA], rv0, semA)
            b0 = pltpu.async_copy(k_hbm.at[sB], rk1, semB)
            b1 = pltpu.async_copy(q_hbm.at[dB], rq1, semB)
            b2 = pltpu.async_copy(v_hbm.at[sB], rv1, semB)
            a0.wait()
            a1.wait()
            a2.wait()
            pltpu.sync_copy(rk0, ok_hbm.at[pl.ds(off, CH)])
            pltpu.sync_copy(rq0, oq_hbm.at[pl.ds(off, CH)])
            pltpu.sync_copy(rv0, ov_hbm.at[pl.ds(off, CH)])
            b0.wait()
            b1.wait()
            b2.wait()
            pltpu.sync_copy(rk1, ok_hbm.at[pl.ds(off + CH, CH)])
            pltpu.sync_copy(rq1, oq_hbm.at[pl.ds(off + CH, CH)])
            pltpu.sync_copy(rv1, ov_hbm.at[pl.ds(off + CH, CH)])

    return gather3


_GATHER3 = _make_gather3()


_QR = 8192
_ACC_R = _QR + 16
_CH = 128
_DR = NHALF // 16


def _make_scatter3():
    n1 = E_RAW // NS // _CH
    n2 = NHALF // NS // _CH
    ZSTR = _ACC_R // NS
    OSTR = _QR // NS
    DSTR = _DR // NS

    def _chunk_off(c, s, i):
        big = c * E_RAW + s * (E_RAW // NS) + i * _CH
        lp = (2 * E_RAW + c * NHALF + s * (NHALF // NS) + (i - n1) * _CH)
        return jnp.where(i < n1, big, lp)

    n_ch = n1 + n2

    @functools.partial(
        pl.kernel,
        mesh=_SC_MESH,
        out_type=[jax.ShapeDtypeStruct((N, C), jnp.float32),
                  jax.ShapeDtypeStruct((N // 16, C), jnp.float32)],
        scratch_types=[
            pltpu.VMEM_SHARED((_ACC_R, C), jnp.float32),
            pltpu.VMEM_SHARED((_DR, C), jnp.float32),
            pltpu.VMEM((_CH,), jnp.int32),
            pltpu.VMEM((_CH,), jnp.int32),
            pltpu.VMEM((_CH,), jnp.int32),
            pltpu.VMEM((_CH, C), jnp.float32),
            pltpu.VMEM((_CH, C), jnp.float32),
            pltpu.SemaphoreType.DMA,
            pltpu.SemaphoreType.DMA,
        ],
    )
    def scat_all(exv_hbm, exs_hbm, dstl_hbm, z_hbm, agg_hbm, den_hbm,
                 acc, accd, idxr, idxA, idxB, bufA, bufB, semA, semB):
        c = lax.axis_index("c")
        s = lax.axis_index("s")

        def sweep_db(con_hbm, mkidx):
            pltpu.async_copy(con_hbm.at[pl.ds(_chunk_off(c, s, 0), _CH)],
                             bufA, semA)

            @pl.loop(0, n_ch // 2)
            def _(i):
                j0 = 2 * i
                offB = _chunk_off(c, s, j0 + 1)
                hb = pltpu.async_copy(con_hbm.at[pl.ds(offB, _CH)], bufB, semB)
                pltpu.make_async_copy(con_hbm.at[pl.ds(0, _CH)], bufA,
                                      semA).wait()
                pltpu.sync_copy(dstl_hbm.at[pl.ds(_chunk_off(c, s, j0), _CH)],
                                idxr)
                mkidx(idxA)
                pltpu.sync_copy(bufA, acc_for(mkidx).at[idxA], add=True)

                @pl.when(j0 + 2 < n_ch)
                def _():
                    pltpu.async_copy(
                        con_hbm.at[pl.ds(_chunk_off(c, s, j0 + 2), _CH)],
                        bufA, semA)

                hb.wait()
                pltpu.sync_copy(dstl_hbm.at[pl.ds(offB, _CH)], idxr)
                mkidx(idxB)
                pltpu.sync_copy(bufB, acc_for(mkidx).at[idxB], add=True)

        def mk_den(dst):
            for t in range(_CH // 16):
                dst[pl.ds(t * 16, 16)] = lax.shift_right_logical(
                    idxr[pl.ds(t * 16, 16)], 4)

        def mk_q0(dst):
            for t in range(_CH // 16):
                v = idxr[pl.ds(t * 16, 16)]
                dst[pl.ds(t * 16, 16)] = jnp.where(v < _QR, v, _QR)

        def mk_q1(dst):
            for t in range(_CH // 16):
                v = idxr[pl.ds(t * 16, 16)]
                lo = v - _QR
                dst[pl.ds(t * 16, 16)] = jnp.where(lo >= 0, lo, _QR)

        def acc_for(mkidx):
            return accd if mkidx is mk_den else acc

        pltpu.sync_copy(z_hbm.at[pl.ds(0, DSTR)], accd.at[pl.ds(s * DSTR, DSTR)])
        plsc.subcore_barrier()
        sweep_db(exs_hbm, mk_den)
        plsc.subcore_barrier()
        pltpu.sync_copy(accd.at[pl.ds(s * DSTR, DSTR)],
                        den_hbm.at[pl.ds(c * _DR + s * DSTR, DSTR)])
        plsc.subcore_barrier()

        for mkidx, q in ((mk_q0, 0), (mk_q1, 1)):
            pltpu.sync_copy(z_hbm, acc.at[pl.ds(s * ZSTR, ZSTR)])
            plsc.subcore_barrier()
            sweep_db(exv_hbm, mkidx)
            plsc.subcore_barrier()
            pltpu.sync_copy(
                acc.at[pl.ds(s * OSTR, OSTR)],
                agg_hbm.at[pl.ds(c * NHALF + q * _QR + s * OSTR, OSTR)])
            plsc.subcore_barrier()

    return scat_all


_SCAT_ALL = _make_scatter3()


def _lnk(x, g, b):
    m = x.mean(-1, keepdims=True)
    v = ((x - m) ** 2).mean(-1, keepdims=True)
    return (x - m) / jnp.sqrt(v + 1e-5) * g + b


def _headmat():
    ch = lax.broadcasted_iota(jnp.int32, (C, NUM_HEADS), 0) // DH
    hh = lax.broadcasted_iota(jnp.int32, (C, NUM_HEADS), 1)
    return (ch == hh).astype(jnp.float32)


def _embed_body(ego_ref, neb_ref, scl_ref, wdt_ref, bd_ref, out_ref):
    i = pl.program_id(0)
    g = i // (NHALF // _BR)
    blk = i % (NHALF // _BR)
    scl = scl_ref[...]
    s0 = jnp.where(g == 0, scl[0, 0], scl[0, 2])
    s1 = jnp.where(g == 0, scl[0, 1], scl[0, 3])
    hw = (blk * _BR + lax.broadcasted_iota(jnp.int32, (_BR, 1), 0)).astype(jnp.float32)
    ii = jnp.floor(hw / W) - (H - 1) / 2.0
    jj = jnp.mod(hw, W) - (W - 1) / 2.0
    d = jnp.sqrt(jnp.square(s0 * ii) + jnp.square(s1 * jj))
    cc = lax.broadcasted_iota(jnp.int32, (1, C), 1)
    ce = ((cc // 2) * 2).astype(jnp.float32)
    div = jnp.exp(-ce * (math.log(10000.0) / C))
    arg = d * div
    pe = jnp.where((cc % 2) == 0, jnp.sin(arg), jnp.cos(arg)) / math.sqrt(C)
    x = jnp.where(g == 0, ego_ref[...], neb_ref[...]).T
    out_ref[...] = x + jax.lax.dot(pe, wdt_ref[...]) + bd_ref[...]


def _embed(ego2, neb2, scl, wdt, bd):
    nb = NHALF // _BR
    return pl.pallas_call(
        _embed_body,
        grid=(2 * nb,),
        in_specs=[
            pl.BlockSpec((C, _BR), lambda i: (0, i % (NHALF // _BR))),
            pl.BlockSpec((C, _BR), lambda i: (0, i % (NHALF // _BR))),
            pl.BlockSpec((1, 8), lambda i: (0, 0)),
            pl.BlockSpec((C, C), lambda i: (0, 0)),
            pl.BlockSpec((1, C), lambda i: (0, 0)),
        ],
        out_specs=pl.BlockSpec((_BR, C), lambda i: (i, 0)),
        out_shape=jax.ShapeDtypeStruct((N, C), jnp.float32),
    )(ego2, neb2, scl, wdt, bd)


def _einit_body(ev_ref, scl_ref, wd1_ref, bd1_ref, wd2_ref, bd2_ref,
                wee_ref, bee_ref, out_ref):
    i = pl.program_id(0)
    nb = E_RAW // _BR
    ev = ev_ref[...]
    scl = scl_ref[...]
    neb_area = scl[0, 0]
    ego_area = scl[0, 1]
    dis = ev[:, 0:1]
    t = jax.lax.dot(dis, wd1_ref[...]) + bd1_ref[...]
    delta = jax.lax.dot(t, wd2_ref[...]) + bd2_ref[...]
    delta = delta[:, 0]
    ddd = delta / (ev[:, 0] + 1e-7)
    v0 = (ev[:, 0] + delta) / SEARCH_RANGE
    ddn = delta ** 2 / neb_area
    v1 = (ev[:, 1] + ddn) / (1.0 + ddn)
    ddn2 = delta ** 2 / ego_area
    v1n = (ev[:, 1] * (neb_area / ego_area) + ddn2) / (1.0 + ddn2)
    v2 = (ev[:, 2] + ddd) / (1.0 + ddd)
    v3 = (ev[:, 3] + ddd) / (1.0 + ddd)
    r1 = jnp.stack([v0, v1, v2, v3], axis=-1)
    r2 = jnp.stack([v0, v1n, v2, -v3], axis=-1)
    ones = jnp.ones((_BR,), jnp.float32)
    zer = jnp.zeros((_BR,), jnp.float32)
    rc = jnp.stack([zer, ones, zer, ones], axis=-1)
    vals = jnp.where(i < nb, r1, jnp.where(i < 2 * nb, r2, rc))
    out_ref[...] = jax.lax.dot(vals, wee_ref[...]) + bee_ref[...]


def _einit(edge_vals, scl, p):
    nb = E_RAW // _BR
    return pl.pallas_call(
        _einit_body,
        grid=(E_TOT // _BR,),
        in_specs=[
            pl.BlockSpec((_BR, 4), lambda i: (i % (E_RAW // _BR), 0)),
            pl.BlockSpec((1, 8), lambda i: (0, 0)),
            pl.BlockSpec((1, 8), lambda i: (0, 0)),
            pl.BlockSpec((1, 8), lambda i: (0, 0)),
            pl.BlockSpec((8, 1), lambda i: (0, 0)),
            pl.BlockSpec((1, 1), lambda i: (0, 0)),
            pl.BlockSpec((4, C), lambda i: (0, 0)),
            pl.BlockSpec((1, C), lambda i: (0, 0)),
        ],
        out_specs=pl.BlockSpec((_BR, C), lambda i: (i, 0)),
        out_shape=jax.ShapeDtypeStruct((E_TOT, C), jnp.float32),
    )(edge_vals, scl, p['Wd1'], p['bd1'].reshape(1, 8), p['Wd2'],
      p['bd2'].reshape(1, 1), p['W_ee'], p['b_ee'].reshape(1, C))


def _qkv_body(h_ref, wq_ref, wk_ref, wv_ref, q_ref, k_ref, v_ref):
    hb = h_ref[...]
    q_ref[...] = jax.lax.dot(hb, wq_ref[...])
    k_ref[...] = jax.lax.dot(hb, wk_ref[...])
    v_ref[...] = jax.lax.dot(hb, wv_ref[...])


def _qkv(h, p):
    return pl.pallas_call(
        _qkv_body,
        grid=(N // _BR,),
        in_specs=[pl.BlockSpec((_BR, C), lambda i: (i, 0))] +
                 [pl.BlockSpec((C, C), lambda i: (0, 0))] * 3,
        out_specs=[pl.BlockSpec((_BR, C), lambda i: (i, 0))] * 3,
        out_shape=[jax.ShapeDtypeStruct((N, C), jnp.float32)] * 3,
    )(h, p['Wq'], p['Wk'], p['Wv'])


def _escore_body(e_ref, we_ref, ks_ref, qd_ref, vs_ref, dl_ref,
                 eout_ref, exv_ref, exs_ref):
    pe = jax.lax.dot(e_ref[...], we_ref[...])
    s = ks_ref[...] * qd_ref[...] * pe * (1.0 / math.sqrt(DH))
    eout_ref[...] = s
    hm = _headmat()
    logits = jnp.clip(jax.lax.dot(s, hm, precision=jax.lax.Precision.HIGHEST),
                      -5.0, 5.0)
    ex = jnp.exp(logits)
    exf = jax.lax.dot(ex, hm.T, precision=jax.lax.Precision.HIGHEST)
    exv_ref[...] = exf * vs_ref[...]
    sh = (jnp.mod(dl_ref[...], 16) ==
          lax.broadcasted_iota(jnp.int32, (1, 16), 1)).astype(jnp.float32)
    cc = lax.broadcasted_iota(jnp.int32, (16, C), 1)
    rr = lax.broadcasted_iota(jnp.int32, (16, C), 0)
    sp = (cc // 8 == rr).astype(jnp.float32)
    cc8 = lax.broadcasted_iota(jnp.int32, (NUM_HEADS, C), 1)
    rr8 = lax.broadcasted_iota(jnp.int32, (NUM_HEADS, C), 0)
    hp = (jnp.mod(cc8, 8) == rr8).astype(jnp.float32)
    exs_ref[...] = (jax.lax.dot(sh, sp, precision=jax.lax.Precision.HIGHEST) *
                    jax.lax.dot(ex, hp, precision=jax.lax.Precision.HIGHEST))


def _escore(e, ksrc, qdst, vsrc, dl2, p):
    return pl.pallas_call(
        _escore_body,
        grid=(E_TOT // _BR,),
        in_specs=[
            pl.BlockSpec((_BR, C), lambda i: (i, 0)),
            pl.BlockSpec((C, C), lambda i: (0, 0)),
            pl.BlockSpec((_BR, C), lambda i: (i, 0)),
            pl.BlockSpec((_BR, C), lambda i: (i, 0)),
            pl.BlockSpec((_BR, C), lambda i: (i, 0)),
            pl.BlockSpec((_BR, 1), lambda i: (i, 0)),
        ],
        out_specs=[pl.BlockSpec((_BR, C), lambda i: (i, 0))] * 3,
        out_shape=[jax.ShapeDtypeStruct((E_TOT, C), jnp.float32)] * 3,
    )(e, p['We'], ksrc, qdst, vsrc, dl2)


def _hupd_body(h_ref, agg_ref, den_ref, wo_ref, bo_ref, g1_ref, b1_ref,
               wf1_ref, bf1_ref, wf2_ref, bf2_ref, g2_ref, b2_ref, out_ref):
    den8 = den_ref[...] + 1e-9
    den = jax.lax.dot(den8, _headmat().T,
                      precision=jax.lax.Precision.HIGHEST)
    hagg = agg_ref[...] / den
    h = h_ref[...]
    h1 = _lnk(h + jax.lax.dot(hagg, wo_ref[...]) + bo_ref[...],
              g1_ref[...], b1_ref[...])
    ff = jnp.maximum(jax.lax.dot(h1, wf1_ref[...]) + bf1_ref[...], 0.0)
    out_ref[...] = _lnk(h1 + jax.lax.dot(ff, wf2_ref[...]) + bf2_ref[...],
                        g2_ref[...], b2_ref[...])


def _hupd(h, agg, denp, p):
    r1 = lambda a: a.reshape(1, -1)
    return pl.pallas_call(
        _hupd_body,
        grid=(N // _BR,),
        in_specs=[
            pl.BlockSpec((_BR, C), lambda i: (i, 0)),
            pl.BlockSpec((_BR, C), lambda i: (i, 0)),
            pl.BlockSpec((_BR, NUM_HEADS), lambda i: (i, 0)),
            pl.BlockSpec((C, C), lambda i: (0, 0)),
            pl.BlockSpec((1, C), lambda i: (0, 0)),
            pl.BlockSpec((1, C), lambda i: (0, 0)),
            pl.BlockSpec((1, C), lambda i: (0, 0)),
            pl.BlockSpec((C, 2 * C), lambda i: (0, 0)),
            pl.BlockSpec((1, 2 * C), lambda i: (0, 0)),
            pl.BlockSpec((2 * C, C), lambda i: (0, 0)),
            pl.BlockSpec((1, C), lambda i: (0, 0)),
            pl.BlockSpec((1, C), lambda i: (0, 0)),
            pl.BlockSpec((1, C), lambda i: (0, 0)),
        ],
        out_specs=pl.BlockSpec((_BR, C), lambda i: (i, 0)),
        out_shape=jax.ShapeDtypeStruct((N, C), jnp.float32),
    )(h, agg, denp, p['Wo'], r1(p['bo']), r1(p['ln1g']), r1(p['ln1b']),
      p['Wf1'], r1(p['bf1']), p['Wf2'], r1(p['bf2']),
      r1(p['ln2g']), r1(p['ln2b']))


def _eupd_body(e_ref, eo_ref, wo_ref, bo_ref, g1_ref, b1_ref,
               wf1_ref, bf1_ref, wf2_ref, bf2_ref, g2_ref, b2_ref, out_ref):
    e1 = _lnk(e_ref[...] + jax.lax.dot(eo_ref[...], wo_ref[...]) + bo_ref[...],
              g1_ref[...], b1_ref[...])
    ff = jnp.maximum(jax.lax.dot(e1, wf1_ref[...]) + bf1_ref[...], 0.0)
    out_ref[...] = _lnk(e1 + jax.lax.dot(ff, wf2_ref[...]) + bf2_ref[...],
                        g2_ref[...], b2_ref[...])


def _eupd(e, eout, p):
    r1 = lambda a: a.reshape(1, -1)
    return pl.pallas_call(
        _eupd_body,
        grid=(E_TOT // _BR,),
        in_specs=[
            pl.BlockSpec((_BR, C), lambda i: (i, 0)),
            pl.BlockSpec((_BR, C), lambda i: (i, 0)),
            pl.BlockSpec((C, C), lambda i: (0, 0)),
            pl.BlockSpec((1, C), lambda i: (0, 0)),
            pl.BlockSpec((1, C), lambda i: (0, 0)),
            pl.BlockSpec((1, C), lambda i: (0, 0)),
            pl.BlockSpec((C, 2 * C), lambda i: (0, 0)),
            pl.BlockSpec((1, 2 * C), lambda i: (0, 0)),
            pl.BlockSpec((2 * C, C), lambda i: (0, 0)),
            pl.BlockSpec((1, C), lambda i: (0, 0)),
            pl.BlockSpec((1, C), lambda i: (0, 0)),
            pl.BlockSpec((1, C), lambda i: (0, 0)),
        ],
        out_specs=pl.BlockSpec((_BR, C), lambda i: (i, 0)),
        out_shape=jax.ShapeDtypeStruct((E_TOT, C), jnp.float32),
    )(e, eout, p['WoE'], r1(p['boE']), r1(p['lnE1g']), r1(p['lnE1b']),
      p['WfE1'], r1(p['bfE1']), p['WfE2'], r1(p['bfE2']),
      r1(p['lnE2g']), r1(p['lnE2b']))


def _out_transpose_body(h_ref, o_ref):
    o_ref[...] = h_ref[...].reshape(8, W, C).transpose(2, 0, 1)


def _gt_layer(h, e, src_g, dst_g, dst_l, dl2, zrows, p):
    q, k, v = _qkv(h, p)
    ksrc, qdst, vsrc = _GATHER3(k, q, v, src_g, dst_g)
    e_out, exv, exs = _escore(e, ksrc, qdst, vsrc, dl2, p)
    agg, denp = _SCAT_ALL(exv, exs, dst_l, zrows)
    e2 = _eupd(e, e_out, p)
    h2 = _hupd(h, agg, denp.reshape(N, NUM_HEADS), p)
    return h2, e2


def kernel(ego, neb, neb_confidence_map, neb_point_cloud_range, edge_index, edge_vals, params):
    neb_pcr = neb_point_cloud_range
    s0e = (PCR[4] - PCR[1]) / H
    s1e = (PCR[3] - PCR[0]) / W
    s0n = (neb_pcr[4] - neb_pcr[1]) / H
    s1n = (neb_pcr[3] - neb_pcr[0]) / W
    neb_area = s0n * s1n
    ego_area = jnp.float32(s0e * s1e)
    scl_embed = jnp.stack([jnp.float32(s0e), jnp.float32(s1e), s0n, s1n,
                           jnp.float32(0), jnp.float32(0), jnp.float32(0),
                           jnp.float32(0)]).reshape(1, 8).astype(jnp.float32)
    scl_einit = jnp.stack([neb_area, ego_area, jnp.float32(0), jnp.float32(0),
                           jnp.float32(0), jnp.float32(0), jnp.float32(0),
                           jnp.float32(0)]).reshape(1, 8).astype(jnp.float32)

    ei0 = edge_index[0].astype(jnp.int32)
    ei1 = edge_index[1].astype(jnp.int32)
    loop = jnp.arange(N, dtype=jnp.int32)
    ar = jnp.arange(NHALF, dtype=jnp.int32)
    src_g = jnp.concatenate([ei0 + NHALF, ei1, loop])
    dst_g = jnp.concatenate([ei1, ei0 + NHALF, loop])
    dst_l = jnp.concatenate([ei1, ei0, ar, ar])
    dl2 = dst_l.reshape(E_TOT, 1)
    zrows = jnp.zeros((_ACC_R // NS, C), jnp.float32)

    h = _embed(ego.reshape(C, NHALF), neb.reshape(C, NHALF), scl_embed,
               params['W_dis'].T, params['b_dis'].reshape(1, C))
    e = _einit(edge_vals, scl_einit, params)
    for l in range(N_LAYERS):
        h, e = _gt_layer(h, e, src_g, dst_g, dst_l, dl2, zrows,
                         params['layers'][l])

    out = pl.pallas_call(
        _out_transpose_body,
        grid=(H // 8,),
        in_specs=[pl.BlockSpec((8 * W, C), lambda i: (i, 0))],
        out_specs=pl.BlockSpec((C, 8, W), lambda i: (0, i, 0)),
        out_shape=jax.ShapeDtypeStruct((C, H, W), jnp.float32),
    )(h)
    return out


# speedup vs baseline: 1.3893x; 1.0085x over previous
import functools
import math

import jax
import jax.numpy as jnp
from jax import lax
from jax.experimental import pallas as pl
from jax.experimental.pallas import tpu as pltpu
from jax.experimental.pallas import tpu_sc as plsc

C = 128
H = 128
W = 128
E_RAW = 131072
NHALF = H * W
N = 2 * H * W
NUM_HEADS = 8
DH = C // NUM_HEADS
N_LAYERS = 2
SEARCH_RANGE = 3.0
PCR = (-140.8, -40.0, -3.0, 140.8, 40.0, 1.0)

NC = 2
NS = 16
NW = NC * NS
E_TOT = 2 * E_RAW + N

_SC_MESH = plsc.VectorSubcoreMesh(core_axis_name="c", subcore_axis_name="s")
_BR = 4096


def _make_gather3():
    per_w = E_TOT // NW
    CH = 128
    n_ch = per_w // CH

    @functools.partial(
        pl.kernel,
        mesh=_SC_MESH,
        out_type=[jax.ShapeDtypeStruct((E_TOT, C), jnp.float32)] * 3,
        scratch_types=[
            pltpu.VMEM((2 * CH,), jnp.int32),
            pltpu.VMEM((2 * CH,), jnp.int32),
            pltpu.VMEM((CH, C), jnp.float32),
            pltpu.VMEM((CH, C), jnp.float32),
            pltpu.VMEM((CH, C), jnp.float32),
            pltpu.VMEM((CH, C), jnp.float32),
            pltpu.VMEM((CH, C), jnp.float32),
            pltpu.VMEM((CH, C), jnp.float32),
            pltpu.SemaphoreType.DMA,
            pltpu.SemaphoreType.DMA,
        ],
    )
    def gather3(k_hbm, q_hbm, v_hbm, src_hbm, dst_hbm,
                ok_hbm, oq_hbm, ov_hbm, idxs, idxd,
                rk0, rq0, rv0, rk1, rq1, rv1, semA, semB):
        wid = lax.axis_index("s") * NC + lax.axis_index("c")
        base = wid * per_w

        @pl.loop(0, n_ch // 2)
        def _(j):
            off = base + 2 * j * CH
            pltpu.sync_copy(src_hbm.at[pl.ds(off, 2 * CH)], idxs)
            pltpu.sync_copy(dst_hbm.at[pl.ds(off, 2 * CH)], idxd)
            sA, dA = idxs.at[pl.ds(0, CH)], idxd.at[pl.ds(0, CH)]
            sB, dB = idxs.at[pl.ds(CH, CH)], idxd.at[pl.ds(CH, CH)]
            a0 = pltpu.async_copy(k_hbm.at[sA], rk0, semA)
            a1 = pltpu.async_copy(q_hbm.at[dA], rq0, semA)
            a2 = pltpu.async_copy(v_hbm.at[sA], rv0, semA)
            b0 = pltpu.async_copy(k_hbm.at[sB], rk1, semB)
            b1 = pltpu.async_copy(q_hbm.at[dB], rq1, semB)
            b2 = pltpu.async_copy(v_hbm.at[sB], rv1, semB)
            a0.wait()
            a1.wait()
            a2.wait()
            pltpu.sync_copy(rk0, ok_hbm.at[pl.ds(off, CH)])
            pltpu.sync_copy(rq0, oq_hbm.at[pl.ds(off, CH)])
            pltpu.sync_copy(rv0, ov_hbm.at[pl.ds(off, CH)])
            b0.wait()
            b1.wait()
            b2.wait()
            pltpu.sync_copy(rk1, ok_hbm.at[pl.ds(off + CH, CH)])
            pltpu.sync_copy(rq1, oq_hbm.at[pl.ds(off + CH, CH)])
            pltpu.sync_copy(rv1, ov_hbm.at[pl.ds(off + CH, CH)])

    return gather3


_GATHER3 = _make_gather3()


_QR = 8192
_ACC_R = _QR + 16
_CH = 128
_DR = NHALF // 16


def _make_scatter3():
    n1 = E_RAW // NS // _CH
    n2 = NHALF // NS // _CH
    ZSTR = _ACC_R // NS
    OSTR = _QR // NS
    DSTR = _DR // NS

    def _chunk_off(c, s, i):
        big = c * E_RAW + s * (E_RAW // NS) + i * _CH
        lp = (2 * E_RAW + c * NHALF + s * (NHALF // NS) + (i - n1) * _CH)
        return jnp.where(i < n1, big, lp)

    n_ch = n1 + n2

    @functools.partial(
        pl.kernel,
        mesh=_SC_MESH,
        out_type=[jax.ShapeDtypeStruct((N, C), jnp.float32),
                  jax.ShapeDtypeStruct((N // 16, C), jnp.float32)],
        scratch_types=[
            pltpu.VMEM_SHARED((_ACC_R, C), jnp.float32),
            pltpu.VMEM_SHARED((_DR, C), jnp.float32),
            pltpu.VMEM((_CH,), jnp.int32),
            pltpu.VMEM((_CH,), jnp.int32),
            pltpu.VMEM((_CH,), jnp.int32),
            pltpu.VMEM((_CH, C), jnp.float32),
            pltpu.VMEM((_CH, C), jnp.float32),
            pltpu.SemaphoreType.DMA,
            pltpu.SemaphoreType.DMA,
        ],
    )
    def scat_all(exv_hbm, exs_hbm, dstl_hbm, z_hbm, agg_hbm, den_hbm,
                 acc, accd, idxr, idxA, idxB, bufA, bufB, semA, semB):
        c = lax.axis_index("c")
        s = lax.axis_index("s")

        def sweep_db(con_hbm, mkidx):
            pltpu.async_copy(con_hbm.at[pl.ds(_chunk_off(c, s, 0), _CH)],
                             bufA, semA)

            @pl.loop(0, n_ch // 2)
            def _(i):
                j0 = 2 * i
                offB = _chunk_off(c, s, j0 + 1)
                hb = pltpu.async_copy(con_hbm.at[pl.ds(offB, _CH)], bufB, semB)
                pltpu.make_async_copy(con_hbm.at[pl.ds(0, _CH)], bufA,
                                      semA).wait()
                pltpu.sync_copy(dstl_hbm.at[pl.ds(_chunk_off(c, s, j0), _CH)],
                                idxr)
                mkidx(idxA)
                pltpu.sync_copy(bufA, acc_for(mkidx).at[idxA], add=True)

                @pl.when(j0 + 2 < n_ch)
                def _():
                    pltpu.async_copy(
                        con_hbm.at[pl.ds(_chunk_off(c, s, j0 + 2), _CH)],
                        bufA, semA)

                hb.wait()
                pltpu.sync_copy(dstl_hbm.at[pl.ds(offB, _CH)], idxr)
                mkidx(idxB)
                pltpu.sync_copy(bufB, acc_for(mkidx).at[idxB], add=True)

        def mk_den(dst):
            for t in range(_CH // 16):
                dst[pl.ds(t * 16, 16)] = lax.shift_right_logical(
                    idxr[pl.ds(t * 16, 16)], 4)

        def mk_q0(dst):
            for t in range(_CH // 16):
                v = idxr[pl.ds(t * 16, 16)]
                dst[pl.ds(t * 16, 16)] = jnp.where(v < _QR, v, _QR)

        def mk_q1(dst):
            for t in range(_CH // 16):
                v = idxr[pl.ds(t * 16, 16)]
                lo = v - _QR
                dst[pl.ds(t * 16, 16)] = jnp.where(lo >= 0, lo, _QR)

        def acc_for(mkidx):
            return accd if mkidx is mk_den else acc

        pltpu.sync_copy(z_hbm.at[pl.ds(0, DSTR)], accd.at[pl.ds(s * DSTR, DSTR)])
        plsc.subcore_barrier()
        sweep_db(exs_hbm, mk_den)
        plsc.subcore_barrier()
        pltpu.sync_copy(accd.at[pl.ds(s * DSTR, DSTR)],
                        den_hbm.at[pl.ds(c * _DR + s * DSTR, DSTR)])
        plsc.subcore_barrier()

        for mkidx, q in ((mk_q0, 0), (mk_q1, 1)):
            pltpu.sync_copy(z_hbm, acc.at[pl.ds(s * ZSTR, ZSTR)])
            plsc.subcore_barrier()
            sweep_db(exv_hbm, mkidx)
            plsc.subcore_barrier()
            pltpu.sync_copy(
                acc.at[pl.ds(s * OSTR, OSTR)],
                agg_hbm.at[pl.ds(c * NHALF + q * _QR + s * OSTR, OSTR)])
            plsc.subcore_barrier()

    return scat_all


_SCAT_ALL = _make_scatter3()


def _lnk(x, g, b):
    m = x.mean(-1, keepdims=True)
    v = ((x - m) ** 2).mean(-1, keepdims=True)
    return (x - m) / jnp.sqrt(v + 1e-5) * g + b


def _headmat():
    ch = lax.broadcasted_iota(jnp.int32, (C, NUM_HEADS), 0) // DH
    hh = lax.broadcasted_iota(jnp.int32, (C, NUM_HEADS), 1)
    return (ch == hh).astype(jnp.float32)


def _embed_body(ego_ref, neb_ref, scl_ref, wdt_ref, bd_ref, out_ref):
    i = pl.program_id(0)
    g = i // (NHALF // _BR)
    blk = i % (NHALF // _BR)
    scl = scl_ref[...]
    s0 = jnp.where(g == 0, scl[0, 0], scl[0, 2])
    s1 = jnp.where(g == 0, scl[0, 1], scl[0, 3])
    hw = (blk * _BR + lax.broadcasted_iota(jnp.int32, (_BR, 1), 0)).astype(jnp.float32)
    ii = jnp.floor(hw / W) - (H - 1) / 2.0
    jj = jnp.mod(hw, W) - (W - 1) / 2.0
    d = jnp.sqrt(jnp.square(s0 * ii) + jnp.square(s1 * jj))
    cc = lax.broadcasted_iota(jnp.int32, (1, C), 1)
    ce = ((cc // 2) * 2).astype(jnp.float32)
    div = jnp.exp(-ce * (math.log(10000.0) / C))
    arg = d * div
    pe = jnp.where((cc % 2) == 0, jnp.sin(arg), jnp.cos(arg)) / math.sqrt(C)
    x = jnp.where(g == 0, ego_ref[...], neb_ref[...]).T
    out_ref[...] = x + jax.lax.dot(pe, wdt_ref[...]) + bd_ref[...]


def _embed(ego2, neb2, scl, wdt, bd):
    nb = NHALF // _BR
    return pl.pallas_call(
        _embed_body,
        grid=(2 * nb,),
        in_specs=[
            pl.BlockSpec((C, _BR), lambda i: (0, i % (NHALF // _BR))),
            pl.BlockSpec((C, _BR), lambda i: (0, i % (NHALF // _BR))),
            pl.BlockSpec((1, 8), lambda i: (0, 0)),
            pl.BlockSpec((C, C), lambda i: (0, 0)),
            pl.BlockSpec((1, C), lambda i: (0, 0)),
        ],
        out_specs=pl.BlockSpec((_BR, C), lambda i: (i, 0)),
        out_shape=jax.ShapeDtypeStruct((N, C), jnp.float32),
    )(ego2, neb2, scl, wdt, bd)


def _einit_body(ev_ref, scl_ref, wd1_ref, bd1_ref, wd2_ref, bd2_ref,
                wee_ref, bee_ref, out_ref):
    i = pl.program_id(0)
    nb = E_RAW // _BR
    ev = ev_ref[...]
    scl = scl_ref[...]
    neb_area = scl[0, 0]
    ego_area = scl[0, 1]
    dis = ev[:, 0:1]
    t = jax.lax.dot(dis, wd1_ref[...]) + bd1_ref[...]
    delta = jax.lax.dot(t, wd2_ref[...]) + bd2_ref[...]
    delta = delta[:, 0]
    ddd = delta / (ev[:, 0] + 1e-7)
    v0 = (ev[:, 0] + delta) / SEARCH_RANGE
    ddn = delta ** 2 / neb_area
    v1 = (ev[:, 1] + ddn) / (1.0 + ddn)
    ddn2 = delta ** 2 / ego_area
    v1n = (ev[:, 1] * (neb_area / ego_area) + ddn2) / (1.0 + ddn2)
    v2 = (ev[:, 2] + ddd) / (1.0 + ddd)
    v3 = (ev[:, 3] + ddd) / (1.0 + ddd)
    r1 = jnp.stack([v0, v1, v2, v3], axis=-1)
    r2 = jnp.stack([v0, v1n, v2, -v3], axis=-1)
    ones = jnp.ones((_BR,), jnp.float32)
    zer = jnp.zeros((_BR,), jnp.float32)
    rc = jnp.stack([zer, ones, zer, ones], axis=-1)
    vals = jnp.where(i < nb, r1, jnp.where(i < 2 * nb, r2, rc))
    out_ref[...] = jax.lax.dot(vals, wee_ref[...]) + bee_ref[...]


def _einit(edge_vals, scl, p):
    nb = E_RAW // _BR
    return pl.pallas_call(
        _einit_body,
        grid=(E_TOT // _BR,),
        in_specs=[
            pl.BlockSpec((_BR, 4), lambda i: (i % (E_RAW // _BR), 0)),
            pl.BlockSpec((1, 8), lambda i: (0, 0)),
            pl.BlockSpec((1, 8), lambda i: (0, 0)),
            pl.BlockSpec((1, 8), lambda i: (0, 0)),
            pl.BlockSpec((8, 1), lambda i: (0, 0)),
            pl.BlockSpec((1, 1), lambda i: (0, 0)),
            pl.BlockSpec((4, C), lambda i: (0, 0)),
            pl.BlockSpec((1, C), lambda i: (0, 0)),
        ],
        out_specs=pl.BlockSpec((_BR, C), lambda i: (i, 0)),
        out_shape=jax.ShapeDtypeStruct((E_TOT, C), jnp.float32),
    )(edge_vals, scl, p['Wd1'], p['bd1'].reshape(1, 8), p['Wd2'],
      p['bd2'].reshape(1, 1), p['W_ee'], p['b_ee'].reshape(1, C))


def _qkv_body(h_ref, wq_ref, wk_ref, wv_ref, q_ref, k_ref, v_ref):
    hb = h_ref[...]
    q_ref[...] = jax.lax.dot(hb, wq_ref[...])
    k_ref[...] = jax.lax.dot(hb, wk_ref[...])
    v_ref[...] = jax.lax.dot(hb, wv_ref[...])


def _qkv(h, p):
    return pl.pallas_call(
        _qkv_body,
        grid=(N // _BR,),
        in_specs=[pl.BlockSpec((_BR, C), lambda i: (i, 0))] +
                 [pl.BlockSpec((C, C), lambda i: (0, 0))] * 3,
        out_specs=[pl.BlockSpec((_BR, C), lambda i: (i, 0))] * 3,
        out_shape=[jax.ShapeDtypeStruct((N, C), jnp.float32)] * 3,
    )(h, p['Wq'], p['Wk'], p['Wv'])


def _escore_body(e_ref, we_ref, ks_ref, qd_ref, vs_ref, dl_ref,
                 eout_ref, exv_ref, exs_ref):
    pe = jax.lax.dot(e_ref[...], we_ref[...])
    s = ks_ref[...] * qd_ref[...] * pe * (1.0 / math.sqrt(DH))
    eout_ref[...] = s
    hm = _headmat()
    logits = jnp.clip(jax.lax.dot(s, hm, precision=jax.lax.Precision.HIGHEST),
                      -5.0, 5.0)
    ex = jnp.exp(logits)
    exf = jax.lax.dot(ex, hm.T, precision=jax.lax.Precision.HIGHEST)
    exv_ref[...] = exf * vs_ref[...]
    sh = (jnp.mod(dl_ref[...], 16) ==
          lax.broadcasted_iota(jnp.int32, (1, 16), 1)).astype(jnp.float32)
    cc = lax.broadcasted_iota(jnp.int32, (16, C), 1)
    rr = lax.broadcasted_iota(jnp.int32, (16, C), 0)
    sp = (cc // 8 == rr).astype(jnp.float32)
    cc8 = lax.broadcasted_iota(jnp.int32, (NUM_HEADS, C), 1)
    rr8 = lax.broadcasted_iota(jnp.int32, (NUM_HEADS, C), 0)
    hp = (jnp.mod(cc8, 8) == rr8).astype(jnp.float32)
    exs_ref[...] = (jax.lax.dot(sh, sp, precision=jax.lax.Precision.HIGHEST) *
                    jax.lax.dot(ex, hp, precision=jax.lax.Precision.HIGHEST))


def _escore(e, ksrc, qdst, vsrc, dl2, p):
    return pl.pallas_call(
        _escore_body,
        grid=(E_TOT // _BR,),
        in_specs=[
            pl.BlockSpec((_BR, C), lambda i: (i, 0)),
            pl.BlockSpec((C, C), lambda i: (0, 0)),
            pl.BlockSpec((_BR, C), lambda i: (i, 0)),
            pl.BlockSpec((_BR, C), lambda i: (i, 0)),
            pl.BlockSpec((_BR, C), lambda i: (i, 0)),
            pl.BlockSpec((_BR, 1), lambda i: (i, 0)),
        ],
        out_specs=[pl.BlockSpec((_BR, C), lambda i: (i, 0))] * 3,
        out_shape=[jax.ShapeDtypeStruct((E_TOT, C), jnp.float32)] * 3,
    )(e, p['We'], ksrc, qdst, vsrc, dl2)


def _hupd_body(h_ref, agg_ref, den_ref, wo_ref, bo_ref, g1_ref, b1_ref,
               wf1_ref, bf1_ref, wf2_ref, bf2_ref, g2_ref, b2_ref, out_ref):
    den8 = den_ref[...] + 1e-9
    den = jax.lax.dot(den8, _headmat().T,
                      precision=jax.lax.Precision.HIGHEST)
    hagg = agg_ref[...] / den
    h = h_ref[...]
    h1 = _lnk(h + jax.lax.dot(hagg, wo_ref[...]) + bo_ref[...],
              g1_ref[...], b1_ref[...])
    ff = jnp.maximum(jax.lax.dot(h1, wf1_ref[...]) + bf1_ref[...], 0.0)
    out_ref[...] = _lnk(h1 + jax.lax.dot(ff, wf2_ref[...]) + bf2_ref[...],
                        g2_ref[...], b2_ref[...])


def _hupd(h, agg, denp, p):
    r1 = lambda a: a.reshape(1, -1)
    return pl.pallas_call(
        _hupd_body,
        grid=(N // _BR,),
        in_specs=[
            pl.BlockSpec((_BR, C), lambda i: (i, 0)),
            pl.BlockSpec((_BR, C), lambda i: (i, 0)),
            pl.BlockSpec((_BR, NUM_HEADS), lambda i: (i, 0)),
            pl.BlockSpec((C, C), lambda i: (0, 0)),
            pl.BlockSpec((1, C), lambda i: (0, 0)),
            pl.BlockSpec((1, C), lambda i: (0, 0)),
            pl.BlockSpec((1, C), lambda i: (0, 0)),
            pl.BlockSpec((C, 2 * C), lambda i: (0, 0)),
            pl.BlockSpec((1, 2 * C), lambda i: (0, 0)),
            pl.BlockSpec((2 * C, C), lambda i: (0, 0)),
            pl.BlockSpec((1, C), lambda i: (0, 0)),
            pl.BlockSpec((1, C), lambda i: (0, 0)),
            pl.BlockSpec((1, C), lambda i: (0, 0)),
        ],
        out_specs=pl.BlockSpec((_BR, C), lambda i: (i, 0)),
        out_shape=jax.ShapeDtypeStruct((N, C), jnp.float32),
    )(h, agg, denp, p['Wo'], r1(p['bo']), r1(p['ln1g']), r1(p['ln1b']),
      p['Wf1'], r1(p['bf1']), p['Wf2'], r1(p['bf2']),
      r1(p['ln2g']), r1(p['ln2b']))


def _eupd_body(e_ref, eo_ref, wo_ref, bo_ref, g1_ref, b1_ref,
               wf1_ref, bf1_ref, wf2_ref, bf2_ref, g2_ref, b2_ref, out_ref):
    e1 = _lnk(e_ref[...] + jax.lax.dot(eo_ref[...], wo_ref[...]) + bo_ref[...],
              g1_ref[...], b1_ref[...])
    ff = jnp.maximum(jax.lax.dot(e1, wf1_ref[...]) + bf1_ref[...], 0.0)
    out_ref[...] = _lnk(e1 + jax.lax.dot(ff, wf2_ref[...]) + bf2_ref[...],
                        g2_ref[...], b2_ref[...])


def _eupd(e, eout, p):
    r1 = lambda a: a.reshape(1, -1)
    return pl.pallas_call(
        _eupd_body,
        grid=(E_TOT // _BR,),
        in_specs=[
            pl.BlockSpec((_BR, C), lambda i: (i, 0)),
            pl.BlockSpec((_BR, C), lambda i: (i, 0)),
            pl.BlockSpec((C, C), lambda i: (0, 0)),
            pl.BlockSpec((1, C), lambda i: (0, 0)),
            pl.BlockSpec((1, C), lambda i: (0, 0)),
            pl.BlockSpec((1, C), lambda i: (0, 0)),
            pl.BlockSpec((C, 2 * C), lambda i: (0, 0)),
            pl.BlockSpec((1, 2 * C), lambda i: (0, 0)),
            pl.BlockSpec((2 * C, C), lambda i: (0, 0)),
            pl.BlockSpec((1, C), lambda i: (0, 0)),
            pl.BlockSpec((1, C), lambda i: (0, 0)),
            pl.BlockSpec((1, C), lambda i: (0, 0)),
        ],
        out_specs=pl.BlockSpec((_BR, C), lambda i: (i, 0)),
        out_shape=jax.ShapeDtypeStruct((E_TOT, C), jnp.float32),
    )(e, eout, p['WoE'], r1(p['boE']), r1(p['lnE1g']), r1(p['lnE1b']),
      p['WfE1'], r1(p['bfE1']), p['WfE2'], r1(p['bfE2']),
      r1(p['lnE2g']), r1(p['lnE2b']))


def _out_transpose_body(h_ref, o_ref):
    o_ref[...] = h_ref[...].reshape(8, W, C).transpose(2, 0, 1)


def _gt_layer(h, e, src_g, dst_g, dst_l, dl2, zrows, p):
    q, k, v = _qkv(h, p)
    ksrc, qdst, vsrc = _GATHER3(k, q, v, src_g, dst_g)
    e_out, exv, exs = _escore(e, ksrc, qdst, vsrc, dl2, p)
    agg, denp = _SCAT_ALL(exv, exs, dst_l, zrows)
    e2 = _eupd(e, e_out, p)
    h2 = _hupd(h, agg, denp.reshape(N, NUM_HEADS), p)
    return h2, e2


def kernel(ego, neb, neb_confidence_map, neb_point_cloud_range, edge_index, edge_vals, params):
    neb_pcr = neb_point_cloud_range
    s0e = (PCR[4] - PCR[1]) / H
    s1e = (PCR[3] - PCR[0]) / W
    s0n = (neb_pcr[4] - neb_pcr[1]) / H
    s1n = (neb_pcr[3] - neb_pcr[0]) / W
    neb_area = s0n * s1n
    ego_area = jnp.float32(s0e * s1e)
    scl_embed = jnp.stack([jnp.float32(s0e), jnp.float32(s1e), s0n, s1n,
                           jnp.float32(0), jnp.float32(0), jnp.float32(0),
                           jnp.float32(0)]).reshape(1, 8).astype(jnp.float32)
    scl_einit = jnp.stack([neb_area, ego_area, jnp.float32(0), jnp.float32(0),
                           jnp.float32(0), jnp.float32(0), jnp.float32(0),
                           jnp.float32(0)]).reshape(1, 8).astype(jnp.float32)

    ei0 = edge_index[0].astype(jnp.int32)
    ei1 = edge_index[1].astype(jnp.int32)
    loop = jnp.arange(N, dtype=jnp.int32)
    ar = jnp.arange(NHALF, dtype=jnp.int32)
    src_g = jnp.concatenate([ei0 + NHALF, ei1, loop])
    dst_g = jnp.concatenate([ei1, ei0 + NHALF, loop])
    dst_l = jnp.concatenate([ei1, ei0, ar, ar])
    dl2 = dst_l.reshape(E_TOT, 1)
    zrows = jnp.zeros((_ACC_R // NS, C), jnp.float32)

    h = _embed(ego.reshape(C, NHALF), neb.reshape(C, NHALF), scl_embed,
               params['W_dis'].T, params['b_dis'].reshape(1, C))
    e = _einit(edge_vals, scl_einit, params)
    for l in range(N_LAYERS):
        h, e = _gt_layer(h, e, src_g, dst_g, dst_l, dl2, zrows,
                         params['layers'][l])

    out = pl.pallas_call(
        _out_transpose_body,
        grid=(H // 8,),
        in_specs=[pl.BlockSpec((8 * W, C), lambda i: (i, 0))],
        out_specs=pl.BlockSpec((C, 8, W), lambda i: (0, i, 0)),
        out_shape=jax.ShapeDtypeStruct((C, H, W), jnp.float32),
    )(h)
    return out
